# baseline (speedup 1.0000x reference)
import functools

import jax
import jax.numpy as jnp
from jax import lax
from jax.experimental import pallas as pl
from jax.experimental.pallas import tpu as pltpu
from jax.experimental.pallas import tpu_sc as plsc

N = 10000
E = 320000
D_IN = 128
H = 64
EMB = 2
NCLS = 4

NC = 2
NS = 16
NW = NC * NS
LANES = 16
BLK = 128
BPW = 80
NBLK = NW * BPW
EPAD = NBLK * BLK
FBLK = NBLK + 104
NACC = 10240
PACK = NACC // 2
RPS = NACC // NS
JUNK = N

NBUF = 4
BPW0 = 148
BPW1 = 2 * BPW - BPW0

_mesh = plsc.VectorSubcoreMesh(core_axis_name="c", subcore_axis_name="s")


@functools.partial(
    pl.kernel,
    out_type=jax.ShapeDtypeStruct((NC, NACC, H), jnp.float32),
    mesh=_mesh,
    compiler_params=pltpu.CompilerParams(use_tc_tiling_on_sc=False),
    scratch_types=[
        pltpu.VMEM((BPW0, BLK), jnp.int32),
        pltpu.VMEM((BPW0, BLK), jnp.int32),
        [pltpu.VMEM((BLK, H), jnp.float32) for _ in range(NBUF)],
        pltpu.VMEM_SHARED((NACC, H), jnp.float32),
        [pltpu.SemaphoreType.DMA for _ in range(NBUF)],
        [pltpu.SemaphoreType.DMA for _ in range(NBUF)],
    ],
)
def _sc_agg(hs_hbm, src_hbm, dst_hbm, out_hbm, sidx, didx, rows, acc, gsem, ssem):
    c = lax.axis_index("c")
    s = lax.axis_index("s")
    start = s * (2 * BPW) + c * BPW0
    nblk = jnp.where(c == 0, BPW0, BPW1)

    def g_start(b, j):
        pltpu.async_copy(hs_hbm.at[sidx.at[b]], rows[j], gsem[j])

    def g_wait(j):
        pltpu.make_async_copy(hs_hbm.at[pl.ds(0, BLK)], rows[j], gsem[j]).wait()

    def s_start(b, j):
        pltpu.async_copy(rows[j], acc.at[didx.at[b]], ssem[j], add=True)

    def s_wait(j):
        pltpu.make_async_copy(rows[j], acc.at[pl.ds(0, BLK)], ssem[j]).wait()

    @pl.loop(0, BLK)
    def _(r):
        @pl.loop(0, H, step=LANES)
        def _(k):
            rows[0][r, pl.ds(k, LANES)] = jnp.zeros((LANES,), jnp.float32)

    @pl.loop(0, RPS // BLK)
    def _(j):
        pltpu.sync_copy(rows[0], acc.at[pl.ds(s * RPS + j * BLK, BLK)])

    pltpu.sync_copy(src_hbm.at[pl.ds(start, BPW0)], sidx)
    pltpu.sync_copy(dst_hbm.at[pl.ds(start, BPW0)], didx)
    plsc.subcore_barrier()

    for j in range(NBUF):
        g_start(j, j)

    @pl.loop(0, nblk - NBUF, step=NBUF)
    def _(b0):
        for j in range(NBUF):
            g_wait(j)
            s_start(b0 + j, j)
        for j in range(NBUF):
            s_wait(j)
            g_start(b0 + NBUF + j, j)

    for j in range(NBUF):
        g_wait(j)
        s_start(nblk - NBUF + j, j)
    for j in range(NBUF):
        s_wait(j)

    plsc.subcore_barrier()
    pltpu.sync_copy(
        acc.at[pl.ds(s * RPS, RPS)],
        out_hbm.at[c].at[pl.ds(s * RPS, RPS)],
    )


@functools.partial(
    pl.kernel,
    out_type=jax.ShapeDtypeStruct((NC, NACC, H), jnp.float32),
    mesh=_mesh,
    compiler_params=pltpu.CompilerParams(use_tc_tiling_on_sc=False),
    scratch_types=[
        pltpu.VMEM((BPW, BLK), jnp.int32),
        pltpu.VMEM((BLK, H), jnp.float32),
        pltpu.VMEM_SHARED((NACC, H), jnp.float32),
        pltpu.SemaphoreType.DMA,
    ],
)
def _sc_hist(dst_hbm, out_hbm, didx, ones, acc, hsem):
    c = lax.axis_index("c")
    s = lax.axis_index("s")
    wid = c * NS + s

    @pl.loop(0, BLK)
    def _(r):
        @pl.loop(0, H, step=LANES)
        def _(k):
            ones[r, pl.ds(k, LANES)] = jnp.zeros((LANES,), jnp.float32)

    @pl.loop(0, RPS // BLK)
    def _(j):
        pltpu.sync_copy(ones, acc.at[pl.ds(s * RPS + j * BLK, BLK)])

    @pl.loop(0, BLK)
    def _(r):
        @pl.loop(0, H, step=LANES)
        def _(k):
            ones[r, pl.ds(k, LANES)] = jnp.full((LANES,), 1.0, jnp.float32)

    pltpu.sync_copy(dst_hbm.at[pl.ds(wid * BPW, BPW)], didx)
    plsc.subcore_barrier()

    @pl.loop(0, BPW)
    def _(b):
        pltpu.async_copy(ones, acc.at[didx.at[b]], hsem, add=True)

    @pl.loop(0, BPW)
    def _(b):
        pltpu.make_async_copy(ones, acc.at[pl.ds(0, BLK)], hsem).wait()

    plsc.subcore_barrier()
    pltpu.sync_copy(
        acc.at[pl.ds(s * RPS, RPS)],
        out_hbm.at[c].at[pl.ds(s * RPS, RPS)],
    )


RB = PACK // 5
GRID = 5


def _k1_body(x_ref, w0_ref, dg_ref, h0_ref, hs0_ref, dis_ref, inv_ref):
    deg = dg_ref[0] + dg_ref[1] + 1.0
    dis = lax.rsqrt(deg)
    inv = 1.0 / deg
    h0 = jnp.dot(x_ref[...], w0_ref[...], preferred_element_type=jnp.float32)
    h0_ref[...] = h0
    hs0_ref[...] = h0 * dis
    dis_ref[...] = dis
    inv_ref[...] = inv


def _tc_prep(xp, W0bd, degp):
    return pl.pallas_call(
        _k1_body,
        grid=(GRID,),
        in_specs=[
            pl.BlockSpec((RB, 2 * D_IN), lambda i: (i, 0)),
            pl.BlockSpec((2 * D_IN, 128), lambda i: (0, 0)),
            pl.BlockSpec((NC, RB, 128), lambda i: (0, i, 0)),
        ],
        out_specs=[pl.BlockSpec((RB, 128), lambda i: (i, 0))] * 4,
        out_shape=[jax.ShapeDtypeStruct((PACK, 128), jnp.float32)] * 4,
    )(xp, W0bd, degp)


def _mid_body(act, a_ref, h_ref, dis_ref, inv_ref, b_ref, w_ref, hn_ref, hsn_ref):
    c = dis_ref[...] * (a_ref[0] + a_ref[1]) + h_ref[...] * inv_ref[...] + b_ref[...]
    if act:
        c = jnp.tanh(c)
    hn = jnp.dot(c, w_ref[...], preferred_element_type=jnp.float32)
    hn_ref[...] = hn
    hsn_ref[...] = hn * dis_ref[...]


def _tc_mid(act, aggp, h, dis, inv, bt, Wbd):
    return pl.pallas_call(
        functools.partial(_mid_body, act),
        grid=(GRID,),
        in_specs=[
            pl.BlockSpec((NC, RB, 128), lambda i: (0, i, 0)),
            pl.BlockSpec((RB, 128), lambda i: (i, 0)),
            pl.BlockSpec((RB, 128), lambda i: (i, 0)),
            pl.BlockSpec((RB, 128), lambda i: (i, 0)),
            pl.BlockSpec((1, 128), lambda i: (0, 0)),
            pl.BlockSpec((128, 128), lambda i: (0, 0)),
        ],
        out_specs=[
            pl.BlockSpec((RB, 128), lambda i: (i, 0)),
            pl.BlockSpec((RB, 128), lambda i: (i, 0)),
        ],
        out_shape=[jax.ShapeDtypeStruct((PACK, 128), jnp.float32)] * 2,
    )(aggp, h, dis, inv, bt, Wbd)


def _k4_body(a_ref, h2_ref, dis_ref, inv_ref, b2_ref, wc_ref, bc_ref,
             out_ref, emb_ref):
    c2 = jnp.tanh(
        dis_ref[...] * (a_ref[0] + a_ref[1])
        + h2_ref[...] * inv_ref[...]
        + b2_ref[...]
    )
    out_ref[...] = (
        jnp.dot(c2, wc_ref[...], preferred_element_type=jnp.float32) + bc_ref[...]
    )
    emb_ref[...] = jnp.concatenate([c2[:, 0:EMB], c2[:, H:H + EMB]], axis=1)


def _tc_final(aggp, h2, dis, inv, b2t, Wcbd, bct):
    return pl.pallas_call(
        _k4_body,
        grid=(GRID,),
        in_specs=[
            pl.BlockSpec((NC, RB, 128), lambda i: (0, i, 0)),
            pl.BlockSpec((RB, 128), lambda i: (i, 0)),
            pl.BlockSpec((RB, 128), lambda i: (i, 0)),
            pl.BlockSpec((RB, 128), lambda i: (i, 0)),
            pl.BlockSpec((1, 128), lambda i: (0, 0)),
            pl.BlockSpec((128, 2 * NCLS), lambda i: (0, 0)),
            pl.BlockSpec((1, 2 * NCLS), lambda i: (0, 0)),
        ],
        out_specs=[
            pl.BlockSpec((RB, 2 * NCLS), lambda i: (i, 0)),
            pl.BlockSpec((RB, 2 * EMB), lambda i: (i, 0)),
        ],
        out_shape=[
            jax.ShapeDtypeStruct((PACK, 2 * NCLS), jnp.float32),
            jax.ShapeDtypeStruct((PACK, 2 * EMB), jnp.float32),
        ],
    )(aggp, h2, dis, inv, b2t, Wcbd, bct)


def _blockdiag(W):
    k, m = W.shape
    z = jnp.zeros((k, m), jnp.float32)
    return jnp.concatenate(
        [jnp.concatenate([W, z], axis=1), jnp.concatenate([z, W], axis=1)], axis=0
    )


def kernel(x, edge_index, W0, b0, W1, b1, W2, b2, Wc, bc):
    ei = edge_index.astype(jnp.int32)
    pad = FBLK * BLK - E
    junk = JUNK + jnp.arange(pad, dtype=jnp.int32) % (NACC - N)
    srcpad = jnp.arange(pad, dtype=jnp.int32) * 79 % N
    src = jnp.concatenate([ei[0], srcpad]).reshape(FBLK, BLK)
    dst = jnp.concatenate([ei[1], junk]).reshape(FBLK, BLK)

    xp = jnp.concatenate(
        [x, jnp.zeros((NACC - N, D_IN), jnp.float32)]
    ).reshape(PACK, 2 * D_IN)

    degp = _sc_hist(dst).reshape(NC, PACK, 128)

    h0, hs0, dis, inv = _tc_prep(xp, _blockdiag(W0), degp)

    a0 = _sc_agg(hs0.reshape(NACC, H), src, dst).reshape(NC, PACK, 128)
    h1, hs1 = _tc_mid(False, a0, h0, dis, inv,
                      jnp.tile(b0, 2).reshape(1, 128), _blockdiag(W1))

    a1 = _sc_agg(hs1.reshape(NACC, H), src, dst).reshape(NC, PACK, 128)
    W2p = jnp.concatenate([W2, jnp.zeros((H, H - EMB), jnp.float32)], axis=1)
    h2, hs2 = _tc_mid(True, a1, h1, dis, inv,
                      jnp.tile(b1, 2).reshape(1, 128), _blockdiag(W2p))

    a2 = _sc_agg(hs2.reshape(NACC, H), src, dst).reshape(NC, PACK, 128)
    b2p = jnp.concatenate([b2, jnp.zeros((H - EMB,), jnp.float32)])
    Wcp = jnp.concatenate([Wc, jnp.zeros((H - EMB, NCLS), jnp.float32)], axis=0)
    out_pk, emb_pk = _tc_final(
        a2, h2, dis, inv,
        jnp.tile(b2p, 2).reshape(1, 128), _blockdiag(Wcp),
        jnp.tile(bc, 2).reshape(1, 2 * NCLS),
    )

    out = out_pk.reshape(NACC, NCLS)[:N]
    emb = emb_pk.reshape(NACC, EMB)[:N]
    return (out, emb)


# speedup vs baseline: 1.0971x; 1.0971x over previous
import functools

import jax
import jax.numpy as jnp
from jax import lax
from jax.experimental import pallas as pl
from jax.experimental.pallas import tpu as pltpu
from jax.experimental.pallas import tpu_sc as plsc

N = 10000
E = 320000
D_IN = 128
H = 64
EMB = 2
NCLS = 4

NC = 2
NS = 16
NW = NC * NS
LANES = 16
BLK = 128
BPW = 80
NBLK = NW * BPW
EPAD = NBLK * BLK
FBLK = NBLK + 104
NACC = 10240
PACK = NACC // 2
RPS = NACC // NS
JUNK = N

NBUF = 4
BPW0 = 128
BPW1 = 2 * BPW - BPW0

_mesh = plsc.VectorSubcoreMesh(core_axis_name="c", subcore_axis_name="s")


@functools.partial(
    pl.kernel,
    out_type=jax.ShapeDtypeStruct((NC, NACC, H), jnp.float32),
    mesh=_mesh,
    compiler_params=pltpu.CompilerParams(use_tc_tiling_on_sc=False),
    scratch_types=[
        pltpu.VMEM((BPW0, BLK), jnp.int32),
        pltpu.VMEM((BPW0, BLK), jnp.int32),
        [pltpu.VMEM((BLK, H), jnp.float32) for _ in range(NBUF)],
        pltpu.VMEM_SHARED((NACC, H), jnp.float32),
        [pltpu.SemaphoreType.DMA for _ in range(NBUF)],
        [pltpu.SemaphoreType.DMA for _ in range(NBUF)],
    ],
)
def _sc_agg(hs_hbm, src_hbm, dst_hbm, out_hbm, sidx, didx, rows, acc, gsem, ssem):
    c = lax.axis_index("c")
    s = lax.axis_index("s")
    start = s * (2 * BPW) + c * BPW0
    nblk = jnp.where(c == 0, BPW0, BPW1)

    def g_start(b, j):
        pltpu.async_copy(hs_hbm.at[sidx.at[b]], rows[j], gsem[j])

    def g_wait(j):
        pltpu.make_async_copy(hs_hbm.at[pl.ds(0, BLK)], rows[j], gsem[j]).wait()

    def s_start(b, j):
        pltpu.async_copy(rows[j], acc.at[didx.at[b]], ssem[j], add=True)

    def s_wait(j):
        pltpu.make_async_copy(rows[j], acc.at[pl.ds(0, BLK)], ssem[j]).wait()

    @pl.loop(0, BLK)
    def _(r):
        @pl.loop(0, H, step=LANES)
        def _(k):
            rows[0][r, pl.ds(k, LANES)] = jnp.zeros((LANES,), jnp.float32)

    @pl.loop(0, RPS // BLK)
    def _(j):
        pltpu.sync_copy(rows[0], acc.at[pl.ds(s * RPS + j * BLK, BLK)])

    pltpu.sync_copy(src_hbm.at[pl.ds(start, BPW0)], sidx)
    pltpu.sync_copy(dst_hbm.at[pl.ds(start, BPW0)], didx)
    plsc.subcore_barrier()

    for j in range(NBUF):
        g_start(j, j)

    @pl.loop(0, nblk - NBUF, step=NBUF)
    def _(b0):
        for j in range(NBUF):
            g_wait(j)
            s_start(b0 + j, j)
        for j in range(NBUF):
            s_wait(j)
            g_start(b0 + NBUF + j, j)

    for j in range(NBUF):
        g_wait(j)
        s_start(nblk - NBUF + j, j)
    for j in range(NBUF):
        s_wait(j)

    plsc.subcore_barrier()
    pltpu.sync_copy(
        acc.at[pl.ds(s * RPS, RPS)],
        out_hbm.at[c].at[pl.ds(s * RPS, RPS)],
    )


@functools.partial(
    pl.kernel,
    out_type=jax.ShapeDtypeStruct((NC, NACC, H), jnp.float32),
    mesh=_mesh,
    compiler_params=pltpu.CompilerParams(use_tc_tiling_on_sc=False),
    scratch_types=[
        pltpu.VMEM((BPW, BLK), jnp.int32),
        pltpu.VMEM((BLK, H), jnp.float32),
        pltpu.VMEM_SHARED((NACC, H), jnp.float32),
        pltpu.SemaphoreType.DMA,
    ],
)
def _sc_hist(dst_hbm, out_hbm, didx, ones, acc, hsem):
    c = lax.axis_index("c")
    s = lax.axis_index("s")
    wid = c * NS + s

    @pl.loop(0, BLK)
    def _(r):
        @pl.loop(0, H, step=LANES)
        def _(k):
            ones[r, pl.ds(k, LANES)] = jnp.zeros((LANES,), jnp.float32)

    @pl.loop(0, RPS // BLK)
    def _(j):
        pltpu.sync_copy(ones, acc.at[pl.ds(s * RPS + j * BLK, BLK)])

    @pl.loop(0, BLK)
    def _(r):
        @pl.loop(0, H, step=LANES)
        def _(k):
            ones[r, pl.ds(k, LANES)] = jnp.full((LANES,), 1.0, jnp.float32)

    pltpu.sync_copy(dst_hbm.at[pl.ds(wid * BPW, BPW)], didx)
    plsc.subcore_barrier()

    @pl.loop(0, BPW)
    def _(b):
        pltpu.async_copy(ones, acc.at[didx.at[b]], hsem, add=True)

    @pl.loop(0, BPW)
    def _(b):
        pltpu.make_async_copy(ones, acc.at[pl.ds(0, BLK)], hsem).wait()

    plsc.subcore_barrier()
    pltpu.sync_copy(
        acc.at[pl.ds(s * RPS, RPS)],
        out_hbm.at[c].at[pl.ds(s * RPS, RPS)],
    )


RB = PACK // 5
GRID = 5


def _k1_body(x_ref, w0_ref, dg_ref, h0_ref, hs0_ref, dis_ref, inv_ref):
    deg = dg_ref[0] + dg_ref[1] + 1.0
    dis = lax.rsqrt(deg)
    inv = 1.0 / deg
    h0 = jnp.dot(x_ref[...], w0_ref[...], preferred_element_type=jnp.float32)
    h0_ref[...] = h0
    hs0_ref[...] = h0 * dis
    dis_ref[...] = dis
    inv_ref[...] = inv


def _tc_prep(xp, W0bd, degp):
    return pl.pallas_call(
        _k1_body,
        grid=(GRID,),
        in_specs=[
            pl.BlockSpec((RB, 2 * D_IN), lambda i: (i, 0)),
            pl.BlockSpec((2 * D_IN, 128), lambda i: (0, 0)),
            pl.BlockSpec((NC, RB, 128), lambda i: (0, i, 0)),
        ],
        out_specs=[pl.BlockSpec((RB, 128), lambda i: (i, 0))] * 4,
        out_shape=[jax.ShapeDtypeStruct((PACK, 128), jnp.float32)] * 4,
    )(xp, W0bd, degp)


def _mid_body(act, a_ref, h_ref, dis_ref, inv_ref, b_ref, w_ref, hn_ref, hsn_ref):
    c = dis_ref[...] * (a_ref[0] + a_ref[1]) + h_ref[...] * inv_ref[...] + b_ref[...]
    if act:
        c = jnp.tanh(c)
    hn = jnp.dot(c, w_ref[...], preferred_element_type=jnp.float32)
    hn_ref[...] = hn
    hsn_ref[...] = hn * dis_ref[...]


def _tc_mid(act, aggp, h, dis, inv, bt, Wbd):
    return pl.pallas_call(
        functools.partial(_mid_body, act),
        grid=(GRID,),
        in_specs=[
            pl.BlockSpec((NC, RB, 128), lambda i: (0, i, 0)),
            pl.BlockSpec((RB, 128), lambda i: (i, 0)),
            pl.BlockSpec((RB, 128), lambda i: (i, 0)),
            pl.BlockSpec((RB, 128), lambda i: (i, 0)),
            pl.BlockSpec((1, 128), lambda i: (0, 0)),
            pl.BlockSpec((128, 128), lambda i: (0, 0)),
        ],
        out_specs=[
            pl.BlockSpec((RB, 128), lambda i: (i, 0)),
            pl.BlockSpec((RB, 128), lambda i: (i, 0)),
        ],
        out_shape=[jax.ShapeDtypeStruct((PACK, 128), jnp.float32)] * 2,
    )(aggp, h, dis, inv, bt, Wbd)


def _k4_body(a_ref, h2_ref, dis_ref, inv_ref, b2_ref, wc_ref, bc_ref,
             out_ref, emb_ref):
    c2 = jnp.tanh(
        dis_ref[...] * (a_ref[0] + a_ref[1])
        + h2_ref[...] * inv_ref[...]
        + b2_ref[...]
    )
    out_ref[...] = (
        jnp.dot(c2, wc_ref[...], preferred_element_type=jnp.float32) + bc_ref[...]
    )
    emb_ref[...] = jnp.concatenate([c2[:, 0:EMB], c2[:, H:H + EMB]], axis=1)


def _tc_final(aggp, h2, dis, inv, b2t, Wcbd, bct):
    return pl.pallas_call(
        _k4_body,
        grid=(GRID,),
        in_specs=[
            pl.BlockSpec((NC, RB, 128), lambda i: (0, i, 0)),
            pl.BlockSpec((RB, 128), lambda i: (i, 0)),
            pl.BlockSpec((RB, 128), lambda i: (i, 0)),
            pl.BlockSpec((RB, 128), lambda i: (i, 0)),
            pl.BlockSpec((1, 128), lambda i: (0, 0)),
            pl.BlockSpec((128, 2 * NCLS), lambda i: (0, 0)),
            pl.BlockSpec((1, 2 * NCLS), lambda i: (0, 0)),
        ],
        out_specs=[
            pl.BlockSpec((RB, 2 * NCLS), lambda i: (i, 0)),
            pl.BlockSpec((RB, 2 * EMB), lambda i: (i, 0)),
        ],
        out_shape=[
            jax.ShapeDtypeStruct((PACK, 2 * NCLS), jnp.float32),
            jax.ShapeDtypeStruct((PACK, 2 * EMB), jnp.float32),
        ],
    )(aggp, h2, dis, inv, b2t, Wcbd, bct)


def _blockdiag(W):
    k, m = W.shape
    z = jnp.zeros((k, m), jnp.float32)
    return jnp.concatenate(
        [jnp.concatenate([W, z], axis=1), jnp.concatenate([z, W], axis=1)], axis=0
    )


def kernel(x, edge_index, W0, b0, W1, b1, W2, b2, Wc, bc):
    ei = edge_index.astype(jnp.int32)
    pad = FBLK * BLK - E
    junk = JUNK + jnp.arange(pad, dtype=jnp.int32) % (NACC - N)
    srcpad = jnp.arange(pad, dtype=jnp.int32) * 79 % N
    src = jnp.concatenate([ei[0], srcpad]).reshape(FBLK, BLK)
    dst = jnp.concatenate([ei[1], junk]).reshape(FBLK, BLK)

    xp = jnp.concatenate(
        [x, jnp.zeros((NACC - N, D_IN), jnp.float32)]
    ).reshape(PACK, 2 * D_IN)

    degp = _sc_hist(dst).reshape(NC, PACK, 128)

    h0, hs0, dis, inv = _tc_prep(xp, _blockdiag(W0), degp)

    a0 = _sc_agg(hs0.reshape(NACC, H), src, dst).reshape(NC, PACK, 128)
    h1, hs1 = _tc_mid(False, a0, h0, dis, inv,
                      jnp.tile(b0, 2).reshape(1, 128), _blockdiag(W1))

    a1 = _sc_agg(hs1.reshape(NACC, H), src, dst).reshape(NC, PACK, 128)
    W2p = jnp.concatenate([W2, jnp.zeros((H, H - EMB), jnp.float32)], axis=1)
    h2, hs2 = _tc_mid(True, a1, h1, dis, inv,
                      jnp.tile(b1, 2).reshape(1, 128), _blockdiag(W2p))

    a2 = _sc_agg(hs2.reshape(NACC, H), src, dst).reshape(NC, PACK, 128)
    b2p = jnp.concatenate([b2, jnp.zeros((H - EMB,), jnp.float32)])
    Wcp = jnp.concatenate([Wc, jnp.zeros((H - EMB, NCLS), jnp.float32)], axis=0)
    out_pk, emb_pk = _tc_final(
        a2, h2, dis, inv,
        jnp.tile(b2p, 2).reshape(1, 128), _blockdiag(Wcp),
        jnp.tile(bc, 2).reshape(1, 2 * NCLS),
    )

    out = out_pk.reshape(NACC, NCLS)[:N]
    emb = emb_pk.reshape(NACC, EMB)[:N]
    return (out, emb)


# speedup vs baseline: 1.1505x; 1.0487x over previous
import functools

import jax
import jax.numpy as jnp
from jax import lax
from jax.experimental import pallas as pl
from jax.experimental.pallas import tpu as pltpu
from jax.experimental.pallas import tpu_sc as plsc

N = 10000
E = 320000
D_IN = 128
H = 64
EMB = 2
NCLS = 4

NC = 2
NS = 16
NW = NC * NS
LANES = 16
BLK = 128
BPW = 80
NBLK = NW * BPW
EPAD = NBLK * BLK
FBLK = NBLK + 104
NACC = 10240
PACK = NACC // 2
RPS = NACC // NS
JUNK = N

NBUF = 4
BPW0 = 128
BPW1 = 2 * BPW - BPW0

_mesh = plsc.VectorSubcoreMesh(core_axis_name="c", subcore_axis_name="s")


@functools.partial(
    pl.kernel,
    out_type=jax.ShapeDtypeStruct((NC, NACC, H), jnp.float32),
    mesh=_mesh,
    compiler_params=pltpu.CompilerParams(use_tc_tiling_on_sc=False),
    scratch_types=[
        pltpu.VMEM((BPW0, BLK), jnp.int32),
        pltpu.VMEM((BPW0, BLK), jnp.int32),
        [pltpu.VMEM((BLK, H), jnp.float32) for _ in range(NBUF)],
        pltpu.VMEM_SHARED((NACC, H), jnp.float32),
        [pltpu.SemaphoreType.DMA for _ in range(NBUF)],
        [pltpu.SemaphoreType.DMA for _ in range(NBUF)],
    ],
)
def _sc_agg(hs_hbm, src_hbm, dst_hbm, out_hbm, sidx, didx, rows, acc, gsem, ssem):
    c = lax.axis_index("c")
    s = lax.axis_index("s")
    start = s * (2 * BPW) + c * BPW0
    nblk = jnp.where(c == 0, BPW0, BPW1)

    def g_start(b, j):
        pltpu.async_copy(hs_hbm.at[sidx.at[b]], rows[j], gsem[j])

    def g_wait(j):
        pltpu.make_async_copy(hs_hbm.at[pl.ds(0, BLK)], rows[j], gsem[j]).wait()

    def s_start(b, j):
        pltpu.async_copy(rows[j], acc.at[didx.at[b]], ssem[j], add=True)

    def s_wait(j):
        pltpu.make_async_copy(rows[j], acc.at[pl.ds(0, BLK)], ssem[j]).wait()

    @pl.loop(0, BLK)
    def _(r):
        @pl.loop(0, H, step=LANES)
        def _(k):
            rows[0][r, pl.ds(k, LANES)] = jnp.zeros((LANES,), jnp.float32)

    @pl.loop(0, RPS // BLK)
    def _(j):
        pltpu.sync_copy(rows[0], acc.at[pl.ds(s * RPS + j * BLK, BLK)])

    pltpu.sync_copy(src_hbm.at[pl.ds(start, BPW0)], sidx)
    pltpu.sync_copy(dst_hbm.at[pl.ds(start, BPW0)], didx)
    plsc.subcore_barrier()

    for j in range(NBUF):
        g_start(j, j)

    @pl.loop(0, nblk - NBUF, step=NBUF)
    def _(b0):
        for j in range(NBUF):
            g_wait(j)
            s_start(b0 + j, j)
        for j in range(NBUF):
            s_wait(j)
            g_start(b0 + NBUF + j, j)

    for j in range(NBUF):
        g_wait(j)
        s_start(nblk - NBUF + j, j)
    for j in range(NBUF):
        s_wait(j)

    plsc.subcore_barrier()
    pltpu.sync_copy(
        acc.at[pl.ds(s * RPS, RPS)],
        out_hbm.at[c].at[pl.ds(s * RPS, RPS)],
    )


@functools.partial(
    pl.kernel,
    out_type=jax.ShapeDtypeStruct((NC, NACC, H), jnp.float32),
    mesh=_mesh,
    compiler_params=pltpu.CompilerParams(use_tc_tiling_on_sc=False),
    scratch_types=[
        pltpu.VMEM((BPW, BLK), jnp.int32),
        pltpu.VMEM((BLK, LANES), jnp.float32),
        pltpu.VMEM((RPS, LANES), jnp.float32),
        pltpu.VMEM((RPS, H), jnp.float32),
        pltpu.VMEM_SHARED((NACC, LANES), jnp.float32),
        pltpu.SemaphoreType.DMA,
    ],
)
def _sc_hist(dst_hbm, out_hbm, didx, ones, t16, t64, acc, hsem):
    c = lax.axis_index("c")
    s = lax.axis_index("s")
    wid = c * NS + s

    @pl.loop(0, BLK)
    def _(r):
        ones[r, pl.ds(0, LANES)] = jnp.zeros((LANES,), jnp.float32)

    @pl.loop(0, RPS // BLK)
    def _(j):
        pltpu.sync_copy(ones, acc.at[pl.ds(s * RPS + j * BLK, BLK)])

    @pl.loop(0, BLK)
    def _(r):
        ones[r, pl.ds(0, LANES)] = jnp.full((LANES,), 1.0, jnp.float32)

    pltpu.sync_copy(dst_hbm.at[pl.ds(wid * BPW, BPW)], didx)
    plsc.subcore_barrier()

    @pl.loop(0, BPW)
    def _(b):
        pltpu.async_copy(ones, acc.at[didx.at[b]], hsem, add=True)

    @pl.loop(0, BPW)
    def _(b):
        pltpu.make_async_copy(ones, acc.at[pl.ds(0, BLK)], hsem).wait()

    plsc.subcore_barrier()
    pltpu.sync_copy(acc.at[pl.ds(s * RPS, RPS)], t16)

    @pl.loop(0, RPS)
    def _(r):
        v = t16[r, pl.ds(0, LANES)]
        @pl.loop(0, H, step=LANES)
        def _(k):
            t64[r, pl.ds(k, LANES)] = v

    pltpu.sync_copy(t64, out_hbm.at[c].at[pl.ds(s * RPS, RPS)])


RB = PACK // 5
GRID = 5


def _k1_body(x_ref, w0_ref, dg_ref, h0_ref, hs0_ref, dis_ref, inv_ref):
    deg = dg_ref[0] + dg_ref[1] + 1.0
    dis = lax.rsqrt(deg)
    inv = 1.0 / deg
    h0 = jnp.dot(x_ref[...], w0_ref[...], preferred_element_type=jnp.float32)
    h0_ref[...] = h0
    hs0_ref[...] = h0 * dis
    dis_ref[...] = dis
    inv_ref[...] = inv


def _tc_prep(xp, W0bd, degp):
    return pl.pallas_call(
        _k1_body,
        grid=(GRID,),
        in_specs=[
            pl.BlockSpec((RB, 2 * D_IN), lambda i: (i, 0)),
            pl.BlockSpec((2 * D_IN, 128), lambda i: (0, 0)),
            pl.BlockSpec((NC, RB, 128), lambda i: (0, i, 0)),
        ],
        out_specs=[pl.BlockSpec((RB, 128), lambda i: (i, 0))] * 4,
        out_shape=[jax.ShapeDtypeStruct((PACK, 128), jnp.float32)] * 4,
    )(xp, W0bd, degp)


def _mid_body(act, a_ref, h_ref, dis_ref, inv_ref, b_ref, w_ref, hn_ref, hsn_ref):
    c = dis_ref[...] * (a_ref[0] + a_ref[1]) + h_ref[...] * inv_ref[...] + b_ref[...]
    if act:
        c = jnp.tanh(c)
    hn = jnp.dot(c, w_ref[...], preferred_element_type=jnp.float32)
    hn_ref[...] = hn
    hsn_ref[...] = hn * dis_ref[...]


def _tc_mid(act, aggp, h, dis, inv, bt, Wbd):
    return pl.pallas_call(
        functools.partial(_mid_body, act),
        grid=(GRID,),
        in_specs=[
            pl.BlockSpec((NC, RB, 128), lambda i: (0, i, 0)),
            pl.BlockSpec((RB, 128), lambda i: (i, 0)),
            pl.BlockSpec((RB, 128), lambda i: (i, 0)),
            pl.BlockSpec((RB, 128), lambda i: (i, 0)),
            pl.BlockSpec((1, 128), lambda i: (0, 0)),
            pl.BlockSpec((128, 128), lambda i: (0, 0)),
        ],
        out_specs=[
            pl.BlockSpec((RB, 128), lambda i: (i, 0)),
            pl.BlockSpec((RB, 128), lambda i: (i, 0)),
        ],
        out_shape=[jax.ShapeDtypeStruct((PACK, 128), jnp.float32)] * 2,
    )(aggp, h, dis, inv, bt, Wbd)


def _k4_body(a_ref, h2_ref, dis_ref, inv_ref, b2_ref, wc_ref, bc_ref,
             out_ref, emb_ref):
    c2 = jnp.tanh(
        dis_ref[...] * (a_ref[0] + a_ref[1])
        + h2_ref[...] * inv_ref[...]
        + b2_ref[...]
    )
    out_ref[...] = (
        jnp.dot(c2, wc_ref[...], preferred_element_type=jnp.float32) + bc_ref[...]
    )
    emb_ref[...] = jnp.concatenate([c2[:, 0:EMB], c2[:, H:H + EMB]], axis=1)


def _tc_final(aggp, h2, dis, inv, b2t, Wcbd, bct):
    return pl.pallas_call(
        _k4_body,
        grid=(GRID,),
        in_specs=[
            pl.BlockSpec((NC, RB, 128), lambda i: (0, i, 0)),
            pl.BlockSpec((RB, 128), lambda i: (i, 0)),
            pl.BlockSpec((RB, 128), lambda i: (i, 0)),
            pl.BlockSpec((RB, 128), lambda i: (i, 0)),
            pl.BlockSpec((1, 128), lambda i: (0, 0)),
            pl.BlockSpec((128, 2 * NCLS), lambda i: (0, 0)),
            pl.BlockSpec((1, 2 * NCLS), lambda i: (0, 0)),
        ],
        out_specs=[
            pl.BlockSpec((RB, 2 * NCLS), lambda i: (i, 0)),
            pl.BlockSpec((RB, 2 * EMB), lambda i: (i, 0)),
        ],
        out_shape=[
            jax.ShapeDtypeStruct((PACK, 2 * NCLS), jnp.float32),
            jax.ShapeDtypeStruct((PACK, 2 * EMB), jnp.float32),
        ],
    )(aggp, h2, dis, inv, b2t, Wcbd, bct)


def _blockdiag(W):
    k, m = W.shape
    z = jnp.zeros((k, m), jnp.float32)
    return jnp.concatenate(
        [jnp.concatenate([W, z], axis=1), jnp.concatenate([z, W], axis=1)], axis=0
    )


def kernel(x, edge_index, W0, b0, W1, b1, W2, b2, Wc, bc):
    ei = edge_index.astype(jnp.int32)
    pad = FBLK * BLK - E
    junk = JUNK + jnp.arange(pad, dtype=jnp.int32) % (NACC - N)
    srcpad = jnp.arange(pad, dtype=jnp.int32) * 79 % N
    src = jnp.concatenate([ei[0], srcpad]).reshape(FBLK, BLK)
    dst = jnp.concatenate([ei[1], junk]).reshape(FBLK, BLK)

    xp = jnp.concatenate(
        [x, jnp.zeros((NACC - N, D_IN), jnp.float32)]
    ).reshape(PACK, 2 * D_IN)

    degp = _sc_hist(dst).reshape(NC, PACK, 128)

    h0, hs0, dis, inv = _tc_prep(xp, _blockdiag(W0), degp)

    a0 = _sc_agg(hs0.reshape(NACC, H), src, dst).reshape(NC, PACK, 128)
    h1, hs1 = _tc_mid(False, a0, h0, dis, inv,
                      jnp.tile(b0, 2).reshape(1, 128), _blockdiag(W1))

    a1 = _sc_agg(hs1.reshape(NACC, H), src, dst).reshape(NC, PACK, 128)
    W2p = jnp.concatenate([W2, jnp.zeros((H, H - EMB), jnp.float32)], axis=1)
    h2, hs2 = _tc_mid(True, a1, h1, dis, inv,
                      jnp.tile(b1, 2).reshape(1, 128), _blockdiag(W2p))

    a2 = _sc_agg(hs2.reshape(NACC, H), src, dst).reshape(NC, PACK, 128)
    b2p = jnp.concatenate([b2, jnp.zeros((H - EMB,), jnp.float32)])
    Wcp = jnp.concatenate([Wc, jnp.zeros((H - EMB, NCLS), jnp.float32)], axis=0)
    out_pk, emb_pk = _tc_final(
        a2, h2, dis, inv,
        jnp.tile(b2p, 2).reshape(1, 128), _blockdiag(Wcp),
        jnp.tile(bc, 2).reshape(1, 2 * NCLS),
    )

    out = out_pk.reshape(NACC, NCLS)[:N]
    emb = emb_pk.reshape(NACC, EMB)[:N]
    return (out, emb)


# speedup vs baseline: 1.2251x; 1.0649x over previous
import functools

import jax
import jax.numpy as jnp
from jax import lax
from jax.experimental import pallas as pl
from jax.experimental.pallas import tpu as pltpu
from jax.experimental.pallas import tpu_sc as plsc

N = 10000
E = 320000
D_IN = 128
H = 64
EMB = 2
NCLS = 4

NC = 2
NS = 16
NW = NC * NS
LANES = 16
BLK = 128
BPW = 80
NBLK = NW * BPW
EPAD = NBLK * BLK
FBLK = NBLK + 104
NACC = 10240
PACK = NACC // 2
RPS = NACC // NS
JUNK = N

NBUF = 4
BPW0 = 116
BPW1 = 2 * BPW - BPW0

_mesh = plsc.VectorSubcoreMesh(core_axis_name="c", subcore_axis_name="s")


@functools.partial(
    pl.kernel,
    out_type=jax.ShapeDtypeStruct((NC, NACC, H), jnp.float32),
    mesh=_mesh,
    compiler_params=pltpu.CompilerParams(use_tc_tiling_on_sc=False),
    scratch_types=[
        pltpu.VMEM((BPW0, BLK), jnp.int32),
        pltpu.VMEM((BPW0, BLK), jnp.int32),
        [pltpu.VMEM((BLK, H), jnp.float32) for _ in range(NBUF)],
        pltpu.VMEM_SHARED((NACC, H), jnp.float32),
        [pltpu.SemaphoreType.DMA for _ in range(NBUF)],
        [pltpu.SemaphoreType.DMA for _ in range(NBUF)],
    ],
)
def _sc_agg(hs_hbm, src_hbm, dst_hbm, out_hbm, sidx, didx, rows, acc, gsem, ssem):
    c = lax.axis_index("c")
    s = lax.axis_index("s")
    start = s * (2 * BPW) + c * BPW0
    nblk = jnp.where(c == 0, BPW0, BPW1)

    def g_start(b, j):
        pltpu.async_copy(hs_hbm.at[sidx.at[b]], rows[j], gsem[j])

    def g_wait(j):
        pltpu.make_async_copy(hs_hbm.at[pl.ds(0, BLK)], rows[j], gsem[j]).wait()

    def s_start(b, j):
        pltpu.async_copy(rows[j], acc.at[didx.at[b]], ssem[j], add=True)

    def s_wait(j):
        pltpu.make_async_copy(rows[j], acc.at[pl.ds(0, BLK)], ssem[j]).wait()

    @pl.loop(0, BLK)
    def _(r):
        @pl.loop(0, H, step=LANES)
        def _(k):
            rows[0][r, pl.ds(k, LANES)] = jnp.zeros((LANES,), jnp.float32)

    @pl.loop(0, RPS // BLK)
    def _(j):
        pltpu.sync_copy(rows[0], acc.at[pl.ds(s * RPS + j * BLK, BLK)])

    pltpu.sync_copy(src_hbm.at[pl.ds(start, BPW0)], sidx)
    pltpu.sync_copy(dst_hbm.at[pl.ds(start, BPW0)], didx)
    plsc.subcore_barrier()

    for j in range(NBUF):
        g_start(j, j)

    @pl.loop(0, nblk - NBUF, step=NBUF)
    def _(b0):
        for j in range(NBUF):
            g_wait(j)
            s_start(b0 + j, j)
        for j in range(NBUF):
            s_wait(j)
            g_start(b0 + NBUF + j, j)

    for j in range(NBUF):
        g_wait(j)
        s_start(nblk - NBUF + j, j)
    for j in range(NBUF):
        s_wait(j)

    plsc.subcore_barrier()
    pltpu.sync_copy(
        acc.at[pl.ds(s * RPS, RPS)],
        out_hbm.at[c].at[pl.ds(s * RPS, RPS)],
    )


@functools.partial(
    pl.kernel,
    out_type=jax.ShapeDtypeStruct((NC, NACC, H), jnp.float32),
    mesh=_mesh,
    compiler_params=pltpu.CompilerParams(use_tc_tiling_on_sc=False),
    scratch_types=[
        pltpu.VMEM((BPW, BLK), jnp.int32),
        pltpu.VMEM((BLK, LANES), jnp.float32),
        pltpu.VMEM((RPS, LANES), jnp.float32),
        pltpu.VMEM((RPS, H), jnp.float32),
        pltpu.VMEM_SHARED((NACC, LANES), jnp.float32),
        pltpu.SemaphoreType.DMA,
    ],
)
def _sc_hist(dst_hbm, out_hbm, didx, ones, t16, t64, acc, hsem):
    c = lax.axis_index("c")
    s = lax.axis_index("s")
    wid = c * NS + s

    @pl.loop(0, BLK)
    def _(r):
        ones[r, pl.ds(0, LANES)] = jnp.zeros((LANES,), jnp.float32)

    @pl.loop(0, RPS // BLK)
    def _(j):
        pltpu.sync_copy(ones, acc.at[pl.ds(s * RPS + j * BLK, BLK)])

    @pl.loop(0, BLK)
    def _(r):
        ones[r, pl.ds(0, LANES)] = jnp.full((LANES,), 1.0, jnp.float32)

    pltpu.sync_copy(dst_hbm.at[pl.ds(wid * BPW, BPW)], didx)
    plsc.subcore_barrier()

    @pl.loop(0, BPW)
    def _(b):
        pltpu.async_copy(ones, acc.at[didx.at[b]], hsem, add=True)

    @pl.loop(0, BPW)
    def _(b):
        pltpu.make_async_copy(ones, acc.at[pl.ds(0, BLK)], hsem).wait()

    plsc.subcore_barrier()
    pltpu.sync_copy(acc.at[pl.ds(s * RPS, RPS)], t16)

    @pl.loop(0, RPS)
    def _(r):
        v = t16[r, pl.ds(0, LANES)]
        @pl.loop(0, H, step=LANES)
        def _(k):
            t64[r, pl.ds(k, LANES)] = v

    pltpu.sync_copy(t64, out_hbm.at[c].at[pl.ds(s * RPS, RPS)])


RB = PACK // 5
GRID = 5


def _k1_body(x_ref, w0_ref, dg_ref, h0_ref, hs0_ref, dis_ref, inv_ref):
    deg = dg_ref[0] + dg_ref[1] + 1.0
    dis = lax.rsqrt(deg)
    inv = 1.0 / deg
    h0 = jnp.dot(x_ref[...], w0_ref[...], preferred_element_type=jnp.float32)
    h0_ref[...] = h0
    hs0_ref[...] = h0 * dis
    dis_ref[...] = dis
    inv_ref[...] = inv


def _tc_prep(xp, W0bd, degp):
    return pl.pallas_call(
        _k1_body,
        grid=(GRID,),
        in_specs=[
            pl.BlockSpec((RB, 2 * D_IN), lambda i: (i, 0)),
            pl.BlockSpec((2 * D_IN, 128), lambda i: (0, 0)),
            pl.BlockSpec((NC, RB, 128), lambda i: (0, i, 0)),
        ],
        out_specs=[pl.BlockSpec((RB, 128), lambda i: (i, 0))] * 4,
        out_shape=[jax.ShapeDtypeStruct((PACK, 128), jnp.float32)] * 4,
    )(xp, W0bd, degp)


def _mid_body(act, a_ref, h_ref, dis_ref, inv_ref, b_ref, w_ref, hn_ref, hsn_ref):
    c = dis_ref[...] * (a_ref[0] + a_ref[1]) + h_ref[...] * inv_ref[...] + b_ref[...]
    if act:
        c = jnp.tanh(c)
    hn = jnp.dot(c, w_ref[...], preferred_element_type=jnp.float32)
    hn_ref[...] = hn
    hsn_ref[...] = hn * dis_ref[...]


def _tc_mid(act, aggp, h, dis, inv, bt, Wbd):
    return pl.pallas_call(
        functools.partial(_mid_body, act),
        grid=(GRID,),
        in_specs=[
            pl.BlockSpec((NC, RB, 128), lambda i: (0, i, 0)),
            pl.BlockSpec((RB, 128), lambda i: (i, 0)),
            pl.BlockSpec((RB, 128), lambda i: (i, 0)),
            pl.BlockSpec((RB, 128), lambda i: (i, 0)),
            pl.BlockSpec((1, 128), lambda i: (0, 0)),
            pl.BlockSpec((128, 128), lambda i: (0, 0)),
        ],
        out_specs=[
            pl.BlockSpec((RB, 128), lambda i: (i, 0)),
            pl.BlockSpec((RB, 128), lambda i: (i, 0)),
        ],
        out_shape=[jax.ShapeDtypeStruct((PACK, 128), jnp.float32)] * 2,
    )(aggp, h, dis, inv, bt, Wbd)


def _k4_body(a_ref, h2_ref, dis_ref, inv_ref, b2_ref, wc_ref, bc_ref,
             out_ref, emb_ref):
    c2 = jnp.tanh(
        dis_ref[...] * (a_ref[0] + a_ref[1])
        + h2_ref[...] * inv_ref[...]
        + b2_ref[...]
    )
    out_ref[...] = (
        jnp.dot(c2, wc_ref[...], preferred_element_type=jnp.float32) + bc_ref[...]
    )
    emb_ref[...] = jnp.concatenate([c2[:, 0:EMB], c2[:, H:H + EMB]], axis=1)


def _tc_final(aggp, h2, dis, inv, b2t, Wcbd, bct):
    return pl.pallas_call(
        _k4_body,
        grid=(GRID,),
        in_specs=[
            pl.BlockSpec((NC, RB, 128), lambda i: (0, i, 0)),
            pl.BlockSpec((RB, 128), lambda i: (i, 0)),
            pl.BlockSpec((RB, 128), lambda i: (i, 0)),
            pl.BlockSpec((RB, 128), lambda i: (i, 0)),
            pl.BlockSpec((1, 128), lambda i: (0, 0)),
            pl.BlockSpec((128, 2 * NCLS), lambda i: (0, 0)),
            pl.BlockSpec((1, 2 * NCLS), lambda i: (0, 0)),
        ],
        out_specs=[
            pl.BlockSpec((RB, 2 * NCLS), lambda i: (i, 0)),
            pl.BlockSpec((RB, 2 * EMB), lambda i: (i, 0)),
        ],
        out_shape=[
            jax.ShapeDtypeStruct((PACK, 2 * NCLS), jnp.float32),
            jax.ShapeDtypeStruct((PACK, 2 * EMB), jnp.float32),
        ],
    )(aggp, h2, dis, inv, b2t, Wcbd, bct)


def _blockdiag(W):
    k, m = W.shape
    z = jnp.zeros((k, m), jnp.float32)
    return jnp.concatenate(
        [jnp.concatenate([W, z], axis=1), jnp.concatenate([z, W], axis=1)], axis=0
    )


def kernel(x, edge_index, W0, b0, W1, b1, W2, b2, Wc, bc):
    ei = edge_index.astype(jnp.int32)
    pad = FBLK * BLK - E
    junk = JUNK + jnp.arange(pad, dtype=jnp.int32) % (NACC - N)
    srcpad = jnp.arange(pad, dtype=jnp.int32) * 79 % N
    src = jnp.concatenate([ei[0], srcpad]).reshape(FBLK, BLK)
    dst = jnp.concatenate([ei[1], junk]).reshape(FBLK, BLK)

    xp = jnp.concatenate(
        [x, jnp.zeros((NACC - N, D_IN), jnp.float32)]
    ).reshape(PACK, 2 * D_IN)

    degp = _sc_hist(dst).reshape(NC, PACK, 128)

    h0, hs0, dis, inv = _tc_prep(xp, _blockdiag(W0), degp)

    a0 = _sc_agg(hs0.reshape(NACC, H), src, dst).reshape(NC, PACK, 128)
    h1, hs1 = _tc_mid(False, a0, h0, dis, inv,
                      jnp.tile(b0, 2).reshape(1, 128), _blockdiag(W1))

    a1 = _sc_agg(hs1.reshape(NACC, H), src, dst).reshape(NC, PACK, 128)
    W2p = jnp.concatenate([W2, jnp.zeros((H, H - EMB), jnp.float32)], axis=1)
    h2, hs2 = _tc_mid(True, a1, h1, dis, inv,
                      jnp.tile(b1, 2).reshape(1, 128), _blockdiag(W2p))

    a2 = _sc_agg(hs2.reshape(NACC, H), src, dst).reshape(NC, PACK, 128)
    b2p = jnp.concatenate([b2, jnp.zeros((H - EMB,), jnp.float32)])
    Wcp = jnp.concatenate([Wc, jnp.zeros((H - EMB, NCLS), jnp.float32)], axis=0)
    out_pk, emb_pk = _tc_final(
        a2, h2, dis, inv,
        jnp.tile(b2p, 2).reshape(1, 128), _blockdiag(Wcp),
        jnp.tile(bc, 2).reshape(1, 2 * NCLS),
    )

    out = out_pk.reshape(NACC, NCLS)[:N]
    emb = emb_pk.reshape(NACC, EMB)[:N]
    return (out, emb)


# speedup vs baseline: 1.2796x; 1.0444x over previous
import functools

import jax
import jax.numpy as jnp
from jax import lax
from jax.experimental import pallas as pl
from jax.experimental.pallas import tpu as pltpu
from jax.experimental.pallas import tpu_sc as plsc

N = 10000
E = 320000
D_IN = 128
H = 64
EMB = 2
NCLS = 4

NC = 2
NS = 16
NW = NC * NS
LANES = 16
BLK = 128
BPW = 80
NBLK = NW * BPW
EPAD = NBLK * BLK
FBLK = NBLK + 104
NACC = 10240
PACK = NACC // 2
RPS = NACC // NS
JUNK = N

NBUF = 4
BPW0 = 108
BPW1 = 2 * BPW - BPW0

_mesh = plsc.VectorSubcoreMesh(core_axis_name="c", subcore_axis_name="s")


@functools.partial(
    pl.kernel,
    out_type=jax.ShapeDtypeStruct((NC, NACC, H), jnp.float32),
    mesh=_mesh,
    compiler_params=pltpu.CompilerParams(use_tc_tiling_on_sc=False),
    scratch_types=[
        pltpu.VMEM((BPW0, BLK), jnp.int32),
        pltpu.VMEM((BPW0, BLK), jnp.int32),
        [pltpu.VMEM((BLK, H), jnp.float32) for _ in range(NBUF)],
        pltpu.VMEM_SHARED((NACC, H), jnp.float32),
        [pltpu.SemaphoreType.DMA for _ in range(NBUF)],
        [pltpu.SemaphoreType.DMA for _ in range(NBUF)],
    ],
)
def _sc_agg(hs_hbm, src_hbm, dst_hbm, out_hbm, sidx, didx, rows, acc, gsem, ssem):
    c = lax.axis_index("c")
    s = lax.axis_index("s")
    start = s * (2 * BPW) + c * BPW0
    nblk = jnp.where(c == 0, BPW0, BPW1)

    def g_start(b, j):
        pltpu.async_copy(hs_hbm.at[sidx.at[b]], rows[j], gsem[j])

    def g_wait(j):
        pltpu.make_async_copy(hs_hbm.at[pl.ds(0, BLK)], rows[j], gsem[j]).wait()

    def s_start(b, j):
        pltpu.async_copy(rows[j], acc.at[didx.at[b]], ssem[j], add=True)

    def s_wait(j):
        pltpu.make_async_copy(rows[j], acc.at[pl.ds(0, BLK)], ssem[j]).wait()

    @pl.loop(0, BLK)
    def _(r):
        @pl.loop(0, H, step=LANES)
        def _(k):
            rows[0][r, pl.ds(k, LANES)] = jnp.zeros((LANES,), jnp.float32)

    @pl.loop(0, RPS // BLK)
    def _(j):
        pltpu.sync_copy(rows[0], acc.at[pl.ds(s * RPS + j * BLK, BLK)])

    pltpu.sync_copy(src_hbm.at[pl.ds(start, BPW0)], sidx)
    pltpu.sync_copy(dst_hbm.at[pl.ds(start, BPW0)], didx)
    plsc.subcore_barrier()

    for j in range(NBUF):
        g_start(j, j)

    @pl.loop(0, nblk - NBUF, step=NBUF)
    def _(b0):
        for j in range(NBUF):
            g_wait(j)
            s_start(b0 + j, j)
        for j in range(NBUF):
            s_wait(j)
            g_start(b0 + NBUF + j, j)

    for j in range(NBUF):
        g_wait(j)
        s_start(nblk - NBUF + j, j)
    for j in range(NBUF):
        s_wait(j)

    plsc.subcore_barrier()
    pltpu.sync_copy(
        acc.at[pl.ds(s * RPS, RPS)],
        out_hbm.at[c].at[pl.ds(s * RPS, RPS)],
    )


@functools.partial(
    pl.kernel,
    out_type=jax.ShapeDtypeStruct((NC, NACC, H), jnp.float32),
    mesh=_mesh,
    compiler_params=pltpu.CompilerParams(use_tc_tiling_on_sc=False),
    scratch_types=[
        pltpu.VMEM((BPW, BLK), jnp.int32),
        pltpu.VMEM((BLK, LANES), jnp.float32),
        pltpu.VMEM((RPS, LANES), jnp.float32),
        pltpu.VMEM((RPS, H), jnp.float32),
        pltpu.VMEM_SHARED((NACC, LANES), jnp.float32),
        pltpu.SemaphoreType.DMA,
    ],
)
def _sc_hist(dst_hbm, out_hbm, didx, ones, t16, t64, acc, hsem):
    c = lax.axis_index("c")
    s = lax.axis_index("s")
    wid = c * NS + s

    @pl.loop(0, BLK)
    def _(r):
        ones[r, pl.ds(0, LANES)] = jnp.zeros((LANES,), jnp.float32)

    @pl.loop(0, RPS // BLK)
    def _(j):
        pltpu.sync_copy(ones, acc.at[pl.ds(s * RPS + j * BLK, BLK)])

    @pl.loop(0, BLK)
    def _(r):
        ones[r, pl.ds(0, LANES)] = jnp.full((LANES,), 1.0, jnp.float32)

    pltpu.sync_copy(dst_hbm.at[pl.ds(wid * BPW, BPW)], didx)
    plsc.subcore_barrier()

    @pl.loop(0, BPW)
    def _(b):
        pltpu.async_copy(ones, acc.at[didx.at[b]], hsem, add=True)

    @pl.loop(0, BPW)
    def _(b):
        pltpu.make_async_copy(ones, acc.at[pl.ds(0, BLK)], hsem).wait()

    plsc.subcore_barrier()
    pltpu.sync_copy(acc.at[pl.ds(s * RPS, RPS)], t16)

    @pl.loop(0, RPS)
    def _(r):
        v = t16[r, pl.ds(0, LANES)]
        @pl.loop(0, H, step=LANES)
        def _(k):
            t64[r, pl.ds(k, LANES)] = v

    pltpu.sync_copy(t64, out_hbm.at[c].at[pl.ds(s * RPS, RPS)])


RB = PACK // 5
GRID = 5


def _k1_body(x_ref, w0_ref, dg_ref, h0_ref, hs0_ref, dis_ref, inv_ref):
    deg = dg_ref[0] + dg_ref[1] + 1.0
    dis = lax.rsqrt(deg)
    inv = 1.0 / deg
    h0 = jnp.dot(x_ref[...], w0_ref[...], preferred_element_type=jnp.float32)
    h0_ref[...] = h0
    hs0_ref[...] = h0 * dis
    dis_ref[...] = dis
    inv_ref[...] = inv


def _tc_prep(xp, W0bd, degp):
    return pl.pallas_call(
        _k1_body,
        grid=(GRID,),
        in_specs=[
            pl.BlockSpec((RB, 2 * D_IN), lambda i: (i, 0)),
            pl.BlockSpec((2 * D_IN, 128), lambda i: (0, 0)),
            pl.BlockSpec((NC, RB, 128), lambda i: (0, i, 0)),
        ],
        out_specs=[pl.BlockSpec((RB, 128), lambda i: (i, 0))] * 4,
        out_shape=[jax.ShapeDtypeStruct((PACK, 128), jnp.float32)] * 4,
    )(xp, W0bd, degp)


def _mid_body(act, a_ref, h_ref, dis_ref, inv_ref, b_ref, w_ref, hn_ref, hsn_ref):
    c = dis_ref[...] * (a_ref[0] + a_ref[1]) + h_ref[...] * inv_ref[...] + b_ref[...]
    if act:
        c = jnp.tanh(c)
    hn = jnp.dot(c, w_ref[...], preferred_element_type=jnp.float32)
    hn_ref[...] = hn
    hsn_ref[...] = hn * dis_ref[...]


def _tc_mid(act, aggp, h, dis, inv, bt, Wbd):
    return pl.pallas_call(
        functools.partial(_mid_body, act),
        grid=(GRID,),
        in_specs=[
            pl.BlockSpec((NC, RB, 128), lambda i: (0, i, 0)),
            pl.BlockSpec((RB, 128), lambda i: (i, 0)),
            pl.BlockSpec((RB, 128), lambda i: (i, 0)),
            pl.BlockSpec((RB, 128), lambda i: (i, 0)),
            pl.BlockSpec((1, 128), lambda i: (0, 0)),
            pl.BlockSpec((128, 128), lambda i: (0, 0)),
        ],
        out_specs=[
            pl.BlockSpec((RB, 128), lambda i: (i, 0)),
            pl.BlockSpec((RB, 128), lambda i: (i, 0)),
        ],
        out_shape=[jax.ShapeDtypeStruct((PACK, 128), jnp.float32)] * 2,
    )(aggp, h, dis, inv, bt, Wbd)


def _k4_body(a_ref, h2_ref, dis_ref, inv_ref, b2_ref, wc_ref, bc_ref,
             out_ref, emb_ref):
    c2 = jnp.tanh(
        dis_ref[...] * (a_ref[0] + a_ref[1])
        + h2_ref[...] * inv_ref[...]
        + b2_ref[...]
    )
    out_ref[...] = (
        jnp.dot(c2, wc_ref[...], preferred_element_type=jnp.float32) + bc_ref[...]
    )
    emb_ref[...] = jnp.concatenate([c2[:, 0:EMB], c2[:, H:H + EMB]], axis=1)


def _tc_final(aggp, h2, dis, inv, b2t, Wcbd, bct):
    return pl.pallas_call(
        _k4_body,
        grid=(GRID,),
        in_specs=[
            pl.BlockSpec((NC, RB, 128), lambda i: (0, i, 0)),
            pl.BlockSpec((RB, 128), lambda i: (i, 0)),
            pl.BlockSpec((RB, 128), lambda i: (i, 0)),
            pl.BlockSpec((RB, 128), lambda i: (i, 0)),
            pl.BlockSpec((1, 128), lambda i: (0, 0)),
            pl.BlockSpec((128, 2 * NCLS), lambda i: (0, 0)),
            pl.BlockSpec((1, 2 * NCLS), lambda i: (0, 0)),
        ],
        out_specs=[
            pl.BlockSpec((RB, 2 * NCLS), lambda i: (i, 0)),
            pl.BlockSpec((RB, 2 * EMB), lambda i: (i, 0)),
        ],
        out_shape=[
            jax.ShapeDtypeStruct((PACK, 2 * NCLS), jnp.float32),
            jax.ShapeDtypeStruct((PACK, 2 * EMB), jnp.float32),
        ],
    )(aggp, h2, dis, inv, b2t, Wcbd, bct)


def _blockdiag(W):
    k, m = W.shape
    z = jnp.zeros((k, m), jnp.float32)
    return jnp.concatenate(
        [jnp.concatenate([W, z], axis=1), jnp.concatenate([z, W], axis=1)], axis=0
    )


def kernel(x, edge_index, W0, b0, W1, b1, W2, b2, Wc, bc):
    ei = edge_index.astype(jnp.int32)
    pad = FBLK * BLK - E
    junk = JUNK + jnp.arange(pad, dtype=jnp.int32) % (NACC - N)
    srcpad = jnp.arange(pad, dtype=jnp.int32) * 79 % N
    src = jnp.concatenate([ei[0], srcpad]).reshape(FBLK, BLK)
    dst = jnp.concatenate([ei[1], junk]).reshape(FBLK, BLK)

    xp = jnp.concatenate(
        [x, jnp.zeros((NACC - N, D_IN), jnp.float32)]
    ).reshape(PACK, 2 * D_IN)

    degp = _sc_hist(dst).reshape(NC, PACK, 128)

    h0, hs0, dis, inv = _tc_prep(xp, _blockdiag(W0), degp)

    a0 = _sc_agg(hs0.reshape(NACC, H), src, dst).reshape(NC, PACK, 128)
    h1, hs1 = _tc_mid(False, a0, h0, dis, inv,
                      jnp.tile(b0, 2).reshape(1, 128), _blockdiag(W1))

    a1 = _sc_agg(hs1.reshape(NACC, H), src, dst).reshape(NC, PACK, 128)
    W2p = jnp.concatenate([W2, jnp.zeros((H, H - EMB), jnp.float32)], axis=1)
    h2, hs2 = _tc_mid(True, a1, h1, dis, inv,
                      jnp.tile(b1, 2).reshape(1, 128), _blockdiag(W2p))

    a2 = _sc_agg(hs2.reshape(NACC, H), src, dst).reshape(NC, PACK, 128)
    b2p = jnp.concatenate([b2, jnp.zeros((H - EMB,), jnp.float32)])
    Wcp = jnp.concatenate([Wc, jnp.zeros((H - EMB, NCLS), jnp.float32)], axis=0)
    out_pk, emb_pk = _tc_final(
        a2, h2, dis, inv,
        jnp.tile(b2p, 2).reshape(1, 128), _blockdiag(Wcp),
        jnp.tile(bc, 2).reshape(1, 2 * NCLS),
    )

    out = out_pk.reshape(NACC, NCLS)[:N]
    emb = emb_pk.reshape(NACC, EMB)[:N]
    return (out, emb)


# speedup vs baseline: 1.3660x; 1.0676x over previous
import functools

import jax
import jax.numpy as jnp
from jax import lax
from jax.experimental import pallas as pl
from jax.experimental.pallas import tpu as pltpu
from jax.experimental.pallas import tpu_sc as plsc

N = 10000
E = 320000
D_IN = 128
H = 64
EMB = 2
NCLS = 4

NC = 2
NS = 16
NW = NC * NS
LANES = 16
BLK = 128
BPW = 80
NBLK = NW * BPW
EPAD = NBLK * BLK
FBLK = NBLK + 104
NACC = 10240
PACK = NACC // 2
RPS = NACC // NS
JUNK = N

NBUF = 4
BPW0 = 96
BPW1 = 2 * BPW - BPW0

_mesh = plsc.VectorSubcoreMesh(core_axis_name="c", subcore_axis_name="s")


@functools.partial(
    pl.kernel,
    out_type=jax.ShapeDtypeStruct((NC, NACC, H), jnp.float32),
    mesh=_mesh,
    compiler_params=pltpu.CompilerParams(use_tc_tiling_on_sc=False),
    scratch_types=[
        pltpu.VMEM((BPW0, BLK), jnp.int32),
        pltpu.VMEM((BPW0, BLK), jnp.int32),
        [pltpu.VMEM((BLK, H), jnp.float32) for _ in range(NBUF)],
        pltpu.VMEM_SHARED((NACC, H), jnp.float32),
        [pltpu.SemaphoreType.DMA for _ in range(NBUF)],
        [pltpu.SemaphoreType.DMA for _ in range(NBUF)],
    ],
)
def _sc_agg(hs_hbm, src_hbm, dst_hbm, out_hbm, sidx, didx, rows, acc, gsem, ssem):
    c = lax.axis_index("c")
    s = lax.axis_index("s")
    start = s * (2 * BPW) + c * BPW0
    nblk = jnp.where(c == 0, BPW0, BPW1)

    def g_start(b, j):
        pltpu.async_copy(hs_hbm.at[sidx.at[b]], rows[j], gsem[j])

    def g_wait(j):
        pltpu.make_async_copy(hs_hbm.at[pl.ds(0, BLK)], rows[j], gsem[j]).wait()

    def s_start(b, j):
        pltpu.async_copy(rows[j], acc.at[didx.at[b]], ssem[j], add=True)

    def s_wait(j):
        pltpu.make_async_copy(rows[j], acc.at[pl.ds(0, BLK)], ssem[j]).wait()

    @pl.loop(0, BLK)
    def _(r):
        @pl.loop(0, H, step=LANES)
        def _(k):
            rows[0][r, pl.ds(k, LANES)] = jnp.zeros((LANES,), jnp.float32)

    @pl.loop(0, RPS // BLK)
    def _(j):
        pltpu.sync_copy(rows[0], acc.at[pl.ds(s * RPS + j * BLK, BLK)])

    pltpu.sync_copy(src_hbm.at[pl.ds(start, BPW0)], sidx)
    pltpu.sync_copy(dst_hbm.at[pl.ds(start, BPW0)], didx)
    plsc.subcore_barrier()

    for j in range(NBUF):
        g_start(j, j)

    @pl.loop(0, nblk - NBUF, step=NBUF)
    def _(b0):
        for j in range(NBUF):
            g_wait(j)
            s_start(b0 + j, j)
        for j in range(NBUF):
            s_wait(j)
            g_start(b0 + NBUF + j, j)

    for j in range(NBUF):
        g_wait(j)
        s_start(nblk - NBUF + j, j)
    for j in range(NBUF):
        s_wait(j)

    plsc.subcore_barrier()
    pltpu.sync_copy(
        acc.at[pl.ds(s * RPS, RPS)],
        out_hbm.at[c].at[pl.ds(s * RPS, RPS)],
    )


@functools.partial(
    pl.kernel,
    out_type=jax.ShapeDtypeStruct((NC, NACC, H), jnp.float32),
    mesh=_mesh,
    compiler_params=pltpu.CompilerParams(use_tc_tiling_on_sc=False),
    scratch_types=[
        pltpu.VMEM((BPW, BLK), jnp.int32),
        pltpu.VMEM((BLK, LANES), jnp.float32),
        pltpu.VMEM((RPS, LANES), jnp.float32),
        pltpu.VMEM((RPS, H), jnp.float32),
        pltpu.VMEM_SHARED((NACC, LANES), jnp.float32),
        pltpu.SemaphoreType.DMA,
    ],
)
def _sc_hist(dst_hbm, out_hbm, didx, ones, t16, t64, acc, hsem):
    c = lax.axis_index("c")
    s = lax.axis_index("s")
    wid = c * NS + s

    @pl.loop(0, BLK)
    def _(r):
        ones[r, pl.ds(0, LANES)] = jnp.zeros((LANES,), jnp.float32)

    @pl.loop(0, RPS // BLK)
    def _(j):
        pltpu.sync_copy(ones, acc.at[pl.ds(s * RPS + j * BLK, BLK)])

    @pl.loop(0, BLK)
    def _(r):
        ones[r, pl.ds(0, LANES)] = jnp.full((LANES,), 1.0, jnp.float32)

    pltpu.sync_copy(dst_hbm.at[pl.ds(wid * BPW, BPW)], didx)
    plsc.subcore_barrier()

    @pl.loop(0, BPW)
    def _(b):
        pltpu.async_copy(ones, acc.at[didx.at[b]], hsem, add=True)

    @pl.loop(0, BPW)
    def _(b):
        pltpu.make_async_copy(ones, acc.at[pl.ds(0, BLK)], hsem).wait()

    plsc.subcore_barrier()
    pltpu.sync_copy(acc.at[pl.ds(s * RPS, RPS)], t16)

    @pl.loop(0, RPS)
    def _(r):
        v = t16[r, pl.ds(0, LANES)]
        @pl.loop(0, H, step=LANES)
        def _(k):
            t64[r, pl.ds(k, LANES)] = v

    pltpu.sync_copy(t64, out_hbm.at[c].at[pl.ds(s * RPS, RPS)])


RB = PACK // 5
GRID = 5


def _k1_body(x_ref, w0_ref, dg_ref, h0_ref, hs0_ref, dis_ref, inv_ref):
    deg = dg_ref[0] + dg_ref[1] + 1.0
    dis = lax.rsqrt(deg)
    inv = 1.0 / deg
    h0 = jnp.dot(x_ref[...], w0_ref[...], preferred_element_type=jnp.float32)
    h0_ref[...] = h0
    hs0_ref[...] = h0 * dis
    dis_ref[...] = dis
    inv_ref[...] = inv


def _tc_prep(xp, W0bd, degp):
    return pl.pallas_call(
        _k1_body,
        grid=(GRID,),
        in_specs=[
            pl.BlockSpec((RB, 2 * D_IN), lambda i: (i, 0)),
            pl.BlockSpec((2 * D_IN, 128), lambda i: (0, 0)),
            pl.BlockSpec((NC, RB, 128), lambda i: (0, i, 0)),
        ],
        out_specs=[pl.BlockSpec((RB, 128), lambda i: (i, 0))] * 4,
        out_shape=[jax.ShapeDtypeStruct((PACK, 128), jnp.float32)] * 4,
    )(xp, W0bd, degp)


def _mid_body(act, a_ref, h_ref, dis_ref, inv_ref, b_ref, w_ref, hn_ref, hsn_ref):
    c = dis_ref[...] * (a_ref[0] + a_ref[1]) + h_ref[...] * inv_ref[...] + b_ref[...]
    if act:
        c = jnp.tanh(c)
    hn = jnp.dot(c, w_ref[...], preferred_element_type=jnp.float32)
    hn_ref[...] = hn
    hsn_ref[...] = hn * dis_ref[...]


def _tc_mid(act, aggp, h, dis, inv, bt, Wbd):
    return pl.pallas_call(
        functools.partial(_mid_body, act),
        grid=(GRID,),
        in_specs=[
            pl.BlockSpec((NC, RB, 128), lambda i: (0, i, 0)),
            pl.BlockSpec((RB, 128), lambda i: (i, 0)),
            pl.BlockSpec((RB, 128), lambda i: (i, 0)),
            pl.BlockSpec((RB, 128), lambda i: (i, 0)),
            pl.BlockSpec((1, 128), lambda i: (0, 0)),
            pl.BlockSpec((128, 128), lambda i: (0, 0)),
        ],
        out_specs=[
            pl.BlockSpec((RB, 128), lambda i: (i, 0)),
            pl.BlockSpec((RB, 128), lambda i: (i, 0)),
        ],
        out_shape=[jax.ShapeDtypeStruct((PACK, 128), jnp.float32)] * 2,
    )(aggp, h, dis, inv, bt, Wbd)


def _k4_body(a_ref, h2_ref, dis_ref, inv_ref, b2_ref, wc_ref, bc_ref,
             out_ref, emb_ref):
    c2 = jnp.tanh(
        dis_ref[...] * (a_ref[0] + a_ref[1])
        + h2_ref[...] * inv_ref[...]
        + b2_ref[...]
    )
    out_ref[...] = (
        jnp.dot(c2, wc_ref[...], preferred_element_type=jnp.float32) + bc_ref[...]
    )
    emb_ref[...] = jnp.concatenate([c2[:, 0:EMB], c2[:, H:H + EMB]], axis=1)


def _tc_final(aggp, h2, dis, inv, b2t, Wcbd, bct):
    return pl.pallas_call(
        _k4_body,
        grid=(GRID,),
        in_specs=[
            pl.BlockSpec((NC, RB, 128), lambda i: (0, i, 0)),
            pl.BlockSpec((RB, 128), lambda i: (i, 0)),
            pl.BlockSpec((RB, 128), lambda i: (i, 0)),
            pl.BlockSpec((RB, 128), lambda i: (i, 0)),
            pl.BlockSpec((1, 128), lambda i: (0, 0)),
            pl.BlockSpec((128, 2 * NCLS), lambda i: (0, 0)),
            pl.BlockSpec((1, 2 * NCLS), lambda i: (0, 0)),
        ],
        out_specs=[
            pl.BlockSpec((RB, 2 * NCLS), lambda i: (i, 0)),
            pl.BlockSpec((RB, 2 * EMB), lambda i: (i, 0)),
        ],
        out_shape=[
            jax.ShapeDtypeStruct((PACK, 2 * NCLS), jnp.float32),
            jax.ShapeDtypeStruct((PACK, 2 * EMB), jnp.float32),
        ],
    )(aggp, h2, dis, inv, b2t, Wcbd, bct)


def _blockdiag(W):
    k, m = W.shape
    z = jnp.zeros((k, m), jnp.float32)
    return jnp.concatenate(
        [jnp.concatenate([W, z], axis=1), jnp.concatenate([z, W], axis=1)], axis=0
    )


def kernel(x, edge_index, W0, b0, W1, b1, W2, b2, Wc, bc):
    ei = edge_index.astype(jnp.int32)
    pad = FBLK * BLK - E
    junk = JUNK + jnp.arange(pad, dtype=jnp.int32) % (NACC - N)
    srcpad = jnp.arange(pad, dtype=jnp.int32) * 79 % N
    src = jnp.concatenate([ei[0], srcpad]).reshape(FBLK, BLK)
    dst = jnp.concatenate([ei[1], junk]).reshape(FBLK, BLK)

    xp = jnp.concatenate(
        [x, jnp.zeros((NACC - N, D_IN), jnp.float32)]
    ).reshape(PACK, 2 * D_IN)

    degp = _sc_hist(dst).reshape(NC, PACK, 128)

    h0, hs0, dis, inv = _tc_prep(xp, _blockdiag(W0), degp)

    a0 = _sc_agg(hs0.reshape(NACC, H), src, dst).reshape(NC, PACK, 128)
    h1, hs1 = _tc_mid(False, a0, h0, dis, inv,
                      jnp.tile(b0, 2).reshape(1, 128), _blockdiag(W1))

    a1 = _sc_agg(hs1.reshape(NACC, H), src, dst).reshape(NC, PACK, 128)
    W2p = jnp.concatenate([W2, jnp.zeros((H, H - EMB), jnp.float32)], axis=1)
    h2, hs2 = _tc_mid(True, a1, h1, dis, inv,
                      jnp.tile(b1, 2).reshape(1, 128), _blockdiag(W2p))

    a2 = _sc_agg(hs2.reshape(NACC, H), src, dst).reshape(NC, PACK, 128)
    b2p = jnp.concatenate([b2, jnp.zeros((H - EMB,), jnp.float32)])
    Wcp = jnp.concatenate([Wc, jnp.zeros((H - EMB, NCLS), jnp.float32)], axis=0)
    out_pk, emb_pk = _tc_final(
        a2, h2, dis, inv,
        jnp.tile(b2p, 2).reshape(1, 128), _blockdiag(Wcp),
        jnp.tile(bc, 2).reshape(1, 2 * NCLS),
    )

    out = out_pk.reshape(NACC, NCLS)[:N]
    emb = emb_pk.reshape(NACC, EMB)[:N]
    return (out, emb)


# speedup vs baseline: 1.4685x; 1.0750x over previous
import functools

import jax
import jax.numpy as jnp
from jax import lax
from jax.experimental import pallas as pl
from jax.experimental.pallas import tpu as pltpu
from jax.experimental.pallas import tpu_sc as plsc

N = 10000
E = 320000
D_IN = 128
H = 64
EMB = 2
NCLS = 4

NC = 2
NS = 16
NW = NC * NS
LANES = 16
BLK = 128
BPW = 80
NBLK = NW * BPW
EPAD = NBLK * BLK
FBLK = NBLK + 104
NACC = 10240
PACK = NACC // 2
RPS = NACC // NS
JUNK = N

NBUF = 4
BPW0 = 84
BPW1 = 2 * BPW - BPW0

_mesh = plsc.VectorSubcoreMesh(core_axis_name="c", subcore_axis_name="s")


@functools.partial(
    pl.kernel,
    out_type=jax.ShapeDtypeStruct((NC, NACC, H), jnp.float32),
    mesh=_mesh,
    compiler_params=pltpu.CompilerParams(use_tc_tiling_on_sc=False),
    scratch_types=[
        pltpu.VMEM((BPW0, BLK), jnp.int32),
        pltpu.VMEM((BPW0, BLK), jnp.int32),
        [pltpu.VMEM((BLK, H), jnp.float32) for _ in range(NBUF)],
        pltpu.VMEM_SHARED((NACC, H), jnp.float32),
        [pltpu.SemaphoreType.DMA for _ in range(NBUF)],
        [pltpu.SemaphoreType.DMA for _ in range(NBUF)],
    ],
)
def _sc_agg(hs_hbm, src_hbm, dst_hbm, out_hbm, sidx, didx, rows, acc, gsem, ssem):
    c = lax.axis_index("c")
    s = lax.axis_index("s")
    start = s * (2 * BPW) + c * BPW0
    nblk = jnp.where(c == 0, BPW0, BPW1)

    def g_start(b, j):
        pltpu.async_copy(hs_hbm.at[sidx.at[b]], rows[j], gsem[j])

    def g_wait(j):
        pltpu.make_async_copy(hs_hbm.at[pl.ds(0, BLK)], rows[j], gsem[j]).wait()

    def s_start(b, j):
        pltpu.async_copy(rows[j], acc.at[didx.at[b]], ssem[j], add=True)

    def s_wait(j):
        pltpu.make_async_copy(rows[j], acc.at[pl.ds(0, BLK)], ssem[j]).wait()

    @pl.loop(0, BLK)
    def _(r):
        @pl.loop(0, H, step=LANES)
        def _(k):
            rows[0][r, pl.ds(k, LANES)] = jnp.zeros((LANES,), jnp.float32)

    @pl.loop(0, RPS // BLK)
    def _(j):
        pltpu.sync_copy(rows[0], acc.at[pl.ds(s * RPS + j * BLK, BLK)])

    pltpu.sync_copy(src_hbm.at[pl.ds(start, BPW0)], sidx)
    pltpu.sync_copy(dst_hbm.at[pl.ds(start, BPW0)], didx)
    plsc.subcore_barrier()

    for j in range(NBUF):
        g_start(j, j)

    @pl.loop(0, nblk - NBUF, step=NBUF)
    def _(b0):
        for j in range(NBUF):
            g_wait(j)
            s_start(b0 + j, j)
        for j in range(NBUF):
            s_wait(j)
            g_start(b0 + NBUF + j, j)

    for j in range(NBUF):
        g_wait(j)
        s_start(nblk - NBUF + j, j)
    for j in range(NBUF):
        s_wait(j)

    plsc.subcore_barrier()
    pltpu.sync_copy(
        acc.at[pl.ds(s * RPS, RPS)],
        out_hbm.at[c].at[pl.ds(s * RPS, RPS)],
    )


@functools.partial(
    pl.kernel,
    out_type=jax.ShapeDtypeStruct((NC, NACC, H), jnp.float32),
    mesh=_mesh,
    compiler_params=pltpu.CompilerParams(use_tc_tiling_on_sc=False),
    scratch_types=[
        pltpu.VMEM((BPW, BLK), jnp.int32),
        pltpu.VMEM((BLK, LANES), jnp.float32),
        pltpu.VMEM((RPS, LANES), jnp.float32),
        pltpu.VMEM((RPS, H), jnp.float32),
        pltpu.VMEM_SHARED((NACC, LANES), jnp.float32),
        pltpu.SemaphoreType.DMA,
    ],
)
def _sc_hist(dst_hbm, out_hbm, didx, ones, t16, t64, acc, hsem):
    c = lax.axis_index("c")
    s = lax.axis_index("s")
    wid = c * NS + s

    @pl.loop(0, BLK)
    def _(r):
        ones[r, pl.ds(0, LANES)] = jnp.zeros((LANES,), jnp.float32)

    @pl.loop(0, RPS // BLK)
    def _(j):
        pltpu.sync_copy(ones, acc.at[pl.ds(s * RPS + j * BLK, BLK)])

    @pl.loop(0, BLK)
    def _(r):
        ones[r, pl.ds(0, LANES)] = jnp.full((LANES,), 1.0, jnp.float32)

    pltpu.sync_copy(dst_hbm.at[pl.ds(wid * BPW, BPW)], didx)
    plsc.subcore_barrier()

    @pl.loop(0, BPW)
    def _(b):
        pltpu.async_copy(ones, acc.at[didx.at[b]], hsem, add=True)

    @pl.loop(0, BPW)
    def _(b):
        pltpu.make_async_copy(ones, acc.at[pl.ds(0, BLK)], hsem).wait()

    plsc.subcore_barrier()
    pltpu.sync_copy(acc.at[pl.ds(s * RPS, RPS)], t16)

    @pl.loop(0, RPS)
    def _(r):
        v = t16[r, pl.ds(0, LANES)]
        @pl.loop(0, H, step=LANES)
        def _(k):
            t64[r, pl.ds(k, LANES)] = v

    pltpu.sync_copy(t64, out_hbm.at[c].at[pl.ds(s * RPS, RPS)])


RB = PACK // 5
GRID = 5


def _k1_body(x_ref, w0_ref, dg_ref, h0_ref, hs0_ref, dis_ref, inv_ref):
    deg = dg_ref[0] + dg_ref[1] + 1.0
    dis = lax.rsqrt(deg)
    inv = 1.0 / deg
    h0 = jnp.dot(x_ref[...], w0_ref[...], preferred_element_type=jnp.float32)
    h0_ref[...] = h0
    hs0_ref[...] = h0 * dis
    dis_ref[...] = dis
    inv_ref[...] = inv


def _tc_prep(xp, W0bd, degp):
    return pl.pallas_call(
        _k1_body,
        grid=(GRID,),
        in_specs=[
            pl.BlockSpec((RB, 2 * D_IN), lambda i: (i, 0)),
            pl.BlockSpec((2 * D_IN, 128), lambda i: (0, 0)),
            pl.BlockSpec((NC, RB, 128), lambda i: (0, i, 0)),
        ],
        out_specs=[pl.BlockSpec((RB, 128), lambda i: (i, 0))] * 4,
        out_shape=[jax.ShapeDtypeStruct((PACK, 128), jnp.float32)] * 4,
    )(xp, W0bd, degp)


def _mid_body(act, a_ref, h_ref, dis_ref, inv_ref, b_ref, w_ref, hn_ref, hsn_ref):
    c = dis_ref[...] * (a_ref[0] + a_ref[1]) + h_ref[...] * inv_ref[...] + b_ref[...]
    if act:
        c = jnp.tanh(c)
    hn = jnp.dot(c, w_ref[...], preferred_element_type=jnp.float32)
    hn_ref[...] = hn
    hsn_ref[...] = hn * dis_ref[...]


def _tc_mid(act, aggp, h, dis, inv, bt, Wbd):
    return pl.pallas_call(
        functools.partial(_mid_body, act),
        grid=(GRID,),
        in_specs=[
            pl.BlockSpec((NC, RB, 128), lambda i: (0, i, 0)),
            pl.BlockSpec((RB, 128), lambda i: (i, 0)),
            pl.BlockSpec((RB, 128), lambda i: (i, 0)),
            pl.BlockSpec((RB, 128), lambda i: (i, 0)),
            pl.BlockSpec((1, 128), lambda i: (0, 0)),
            pl.BlockSpec((128, 128), lambda i: (0, 0)),
        ],
        out_specs=[
            pl.BlockSpec((RB, 128), lambda i: (i, 0)),
            pl.BlockSpec((RB, 128), lambda i: (i, 0)),
        ],
        out_shape=[jax.ShapeDtypeStruct((PACK, 128), jnp.float32)] * 2,
    )(aggp, h, dis, inv, bt, Wbd)


def _k4_body(a_ref, h2_ref, dis_ref, inv_ref, b2_ref, wc_ref, bc_ref,
             out_ref, emb_ref):
    c2 = jnp.tanh(
        dis_ref[...] * (a_ref[0] + a_ref[1])
        + h2_ref[...] * inv_ref[...]
        + b2_ref[...]
    )
    out_ref[...] = (
        jnp.dot(c2, wc_ref[...], preferred_element_type=jnp.float32) + bc_ref[...]
    )
    emb_ref[...] = jnp.concatenate([c2[:, 0:EMB], c2[:, H:H + EMB]], axis=1)


def _tc_final(aggp, h2, dis, inv, b2t, Wcbd, bct):
    return pl.pallas_call(
        _k4_body,
        grid=(GRID,),
        in_specs=[
            pl.BlockSpec((NC, RB, 128), lambda i: (0, i, 0)),
            pl.BlockSpec((RB, 128), lambda i: (i, 0)),
            pl.BlockSpec((RB, 128), lambda i: (i, 0)),
            pl.BlockSpec((RB, 128), lambda i: (i, 0)),
            pl.BlockSpec((1, 128), lambda i: (0, 0)),
            pl.BlockSpec((128, 2 * NCLS), lambda i: (0, 0)),
            pl.BlockSpec((1, 2 * NCLS), lambda i: (0, 0)),
        ],
        out_specs=[
            pl.BlockSpec((RB, 2 * NCLS), lambda i: (i, 0)),
            pl.BlockSpec((RB, 2 * EMB), lambda i: (i, 0)),
        ],
        out_shape=[
            jax.ShapeDtypeStruct((PACK, 2 * NCLS), jnp.float32),
            jax.ShapeDtypeStruct((PACK, 2 * EMB), jnp.float32),
        ],
    )(aggp, h2, dis, inv, b2t, Wcbd, bct)


def _blockdiag(W):
    k, m = W.shape
    z = jnp.zeros((k, m), jnp.float32)
    return jnp.concatenate(
        [jnp.concatenate([W, z], axis=1), jnp.concatenate([z, W], axis=1)], axis=0
    )


def kernel(x, edge_index, W0, b0, W1, b1, W2, b2, Wc, bc):
    ei = edge_index.astype(jnp.int32)
    pad = FBLK * BLK - E
    junk = JUNK + jnp.arange(pad, dtype=jnp.int32) % (NACC - N)
    srcpad = jnp.arange(pad, dtype=jnp.int32) * 79 % N
    src = jnp.concatenate([ei[0], srcpad]).reshape(FBLK, BLK)
    dst = jnp.concatenate([ei[1], junk]).reshape(FBLK, BLK)

    xp = jnp.concatenate(
        [x, jnp.zeros((NACC - N, D_IN), jnp.float32)]
    ).reshape(PACK, 2 * D_IN)

    degp = _sc_hist(dst).reshape(NC, PACK, 128)

    h0, hs0, dis, inv = _tc_prep(xp, _blockdiag(W0), degp)

    a0 = _sc_agg(hs0.reshape(NACC, H), src, dst).reshape(NC, PACK, 128)
    h1, hs1 = _tc_mid(False, a0, h0, dis, inv,
                      jnp.tile(b0, 2).reshape(1, 128), _blockdiag(W1))

    a1 = _sc_agg(hs1.reshape(NACC, H), src, dst).reshape(NC, PACK, 128)
    W2p = jnp.concatenate([W2, jnp.zeros((H, H - EMB), jnp.float32)], axis=1)
    h2, hs2 = _tc_mid(True, a1, h1, dis, inv,
                      jnp.tile(b1, 2).reshape(1, 128), _blockdiag(W2p))

    a2 = _sc_agg(hs2.reshape(NACC, H), src, dst).reshape(NC, PACK, 128)
    b2p = jnp.concatenate([b2, jnp.zeros((H - EMB,), jnp.float32)])
    Wcp = jnp.concatenate([Wc, jnp.zeros((H - EMB, NCLS), jnp.float32)], axis=0)
    out_pk, emb_pk = _tc_final(
        a2, h2, dis, inv,
        jnp.tile(b2p, 2).reshape(1, 128), _blockdiag(Wcp),
        jnp.tile(bc, 2).reshape(1, 2 * NCLS),
    )

    out = out_pk.reshape(NACC, NCLS)[:N]
    emb = emb_pk.reshape(NACC, EMB)[:N]
    return (out, emb)


# speedup vs baseline: 1.4987x; 1.0206x over previous
import functools

import jax
import jax.numpy as jnp
from jax import lax
from jax.experimental import pallas as pl
from jax.experimental.pallas import tpu as pltpu
from jax.experimental.pallas import tpu_sc as plsc

N = 10000
E = 320000
D_IN = 128
H = 64
EMB = 2
NCLS = 4

NC = 2
NS = 16
NW = NC * NS
LANES = 16
BLK = 128
BPW = 80
NBLK = NW * BPW
EPAD = NBLK * BLK
FBLK = NBLK + 104
NACC = 10240
PACK = NACC // 2
RPS = NACC // NS
JUNK = N

NBUF = 4
BPW0 = 80
BPW1 = 2 * BPW - BPW0

_mesh = plsc.VectorSubcoreMesh(core_axis_name="c", subcore_axis_name="s")


@functools.partial(
    pl.kernel,
    out_type=jax.ShapeDtypeStruct((NC, NACC, H), jnp.float32),
    mesh=_mesh,
    compiler_params=pltpu.CompilerParams(use_tc_tiling_on_sc=False),
    scratch_types=[
        pltpu.VMEM((BPW0, BLK), jnp.int32),
        pltpu.VMEM((BPW0, BLK), jnp.int32),
        [pltpu.VMEM((BLK, H), jnp.float32) for _ in range(NBUF)],
        pltpu.VMEM_SHARED((NACC, H), jnp.float32),
        [pltpu.SemaphoreType.DMA for _ in range(NBUF)],
        [pltpu.SemaphoreType.DMA for _ in range(NBUF)],
    ],
)
def _sc_agg(hs_hbm, src_hbm, dst_hbm, out_hbm, sidx, didx, rows, acc, gsem, ssem):
    c = lax.axis_index("c")
    s = lax.axis_index("s")
    start = s * (2 * BPW) + c * BPW0
    nblk = jnp.where(c == 0, BPW0, BPW1)

    def g_start(b, j):
        pltpu.async_copy(hs_hbm.at[sidx.at[b]], rows[j], gsem[j])

    def g_wait(j):
        pltpu.make_async_copy(hs_hbm.at[pl.ds(0, BLK)], rows[j], gsem[j]).wait()

    def s_start(b, j):
        pltpu.async_copy(rows[j], acc.at[didx.at[b]], ssem[j], add=True)

    def s_wait(j):
        pltpu.make_async_copy(rows[j], acc.at[pl.ds(0, BLK)], ssem[j]).wait()

    @pl.loop(0, BLK)
    def _(r):
        @pl.loop(0, H, step=LANES)
        def _(k):
            rows[0][r, pl.ds(k, LANES)] = jnp.zeros((LANES,), jnp.float32)

    @pl.loop(0, RPS // BLK)
    def _(j):
        pltpu.sync_copy(rows[0], acc.at[pl.ds(s * RPS + j * BLK, BLK)])

    pltpu.sync_copy(src_hbm.at[pl.ds(start, BPW0)], sidx)
    pltpu.sync_copy(dst_hbm.at[pl.ds(start, BPW0)], didx)
    plsc.subcore_barrier()

    for j in range(NBUF):
        g_start(j, j)

    @pl.loop(0, nblk - NBUF, step=NBUF)
    def _(b0):
        for j in range(NBUF):
            g_wait(j)
            s_start(b0 + j, j)
        for j in range(NBUF):
            s_wait(j)
            g_start(b0 + NBUF + j, j)

    for j in range(NBUF):
        g_wait(j)
        s_start(nblk - NBUF + j, j)
    for j in range(NBUF):
        s_wait(j)

    plsc.subcore_barrier()
    pltpu.sync_copy(
        acc.at[pl.ds(s * RPS, RPS)],
        out_hbm.at[c].at[pl.ds(s * RPS, RPS)],
    )


@functools.partial(
    pl.kernel,
    out_type=jax.ShapeDtypeStruct((NC, NACC, H), jnp.float32),
    mesh=_mesh,
    compiler_params=pltpu.CompilerParams(use_tc_tiling_on_sc=False),
    scratch_types=[
        pltpu.VMEM((BPW, BLK), jnp.int32),
        pltpu.VMEM((BLK, LANES), jnp.float32),
        pltpu.VMEM((RPS, LANES), jnp.float32),
        pltpu.VMEM((RPS, H), jnp.float32),
        pltpu.VMEM_SHARED((NACC, LANES), jnp.float32),
        pltpu.SemaphoreType.DMA,
    ],
)
def _sc_hist(dst_hbm, out_hbm, didx, ones, t16, t64, acc, hsem):
    c = lax.axis_index("c")
    s = lax.axis_index("s")
    wid = c * NS + s

    @pl.loop(0, BLK)
    def _(r):
        ones[r, pl.ds(0, LANES)] = jnp.zeros((LANES,), jnp.float32)

    @pl.loop(0, RPS // BLK)
    def _(j):
        pltpu.sync_copy(ones, acc.at[pl.ds(s * RPS + j * BLK, BLK)])

    @pl.loop(0, BLK)
    def _(r):
        ones[r, pl.ds(0, LANES)] = jnp.full((LANES,), 1.0, jnp.float32)

    pltpu.sync_copy(dst_hbm.at[pl.ds(wid * BPW, BPW)], didx)
    plsc.subcore_barrier()

    @pl.loop(0, BPW)
    def _(b):
        pltpu.async_copy(ones, acc.at[didx.at[b]], hsem, add=True)

    @pl.loop(0, BPW)
    def _(b):
        pltpu.make_async_copy(ones, acc.at[pl.ds(0, BLK)], hsem).wait()

    plsc.subcore_barrier()
    pltpu.sync_copy(acc.at[pl.ds(s * RPS, RPS)], t16)

    @pl.loop(0, RPS)
    def _(r):
        v = t16[r, pl.ds(0, LANES)]
        @pl.loop(0, H, step=LANES)
        def _(k):
            t64[r, pl.ds(k, LANES)] = v

    pltpu.sync_copy(t64, out_hbm.at[c].at[pl.ds(s * RPS, RPS)])


RB = PACK // 5
GRID = 5


def _k1_body(x_ref, w0_ref, dg_ref, h0_ref, hs0_ref, dis_ref, inv_ref):
    deg = dg_ref[0] + dg_ref[1] + 1.0
    dis = lax.rsqrt(deg)
    inv = 1.0 / deg
    h0 = jnp.dot(x_ref[...], w0_ref[...], preferred_element_type=jnp.float32)
    h0_ref[...] = h0
    hs0_ref[...] = h0 * dis
    dis_ref[...] = dis
    inv_ref[...] = inv


def _tc_prep(xp, W0bd, degp):
    return pl.pallas_call(
        _k1_body,
        grid=(GRID,),
        in_specs=[
            pl.BlockSpec((RB, 2 * D_IN), lambda i: (i, 0)),
            pl.BlockSpec((2 * D_IN, 128), lambda i: (0, 0)),
            pl.BlockSpec((NC, RB, 128), lambda i: (0, i, 0)),
        ],
        out_specs=[pl.BlockSpec((RB, 128), lambda i: (i, 0))] * 4,
        out_shape=[jax.ShapeDtypeStruct((PACK, 128), jnp.float32)] * 4,
    )(xp, W0bd, degp)


def _mid_body(act, a_ref, h_ref, dis_ref, inv_ref, b_ref, w_ref, hn_ref, hsn_ref):
    c = dis_ref[...] * (a_ref[0] + a_ref[1]) + h_ref[...] * inv_ref[...] + b_ref[...]
    if act:
        c = jnp.tanh(c)
    hn = jnp.dot(c, w_ref[...], preferred_element_type=jnp.float32)
    hn_ref[...] = hn
    hsn_ref[...] = hn * dis_ref[...]


def _tc_mid(act, aggp, h, dis, inv, bt, Wbd):
    return pl.pallas_call(
        functools.partial(_mid_body, act),
        grid=(GRID,),
        in_specs=[
            pl.BlockSpec((NC, RB, 128), lambda i: (0, i, 0)),
            pl.BlockSpec((RB, 128), lambda i: (i, 0)),
            pl.BlockSpec((RB, 128), lambda i: (i, 0)),
            pl.BlockSpec((RB, 128), lambda i: (i, 0)),
            pl.BlockSpec((1, 128), lambda i: (0, 0)),
            pl.BlockSpec((128, 128), lambda i: (0, 0)),
        ],
        out_specs=[
            pl.BlockSpec((RB, 128), lambda i: (i, 0)),
            pl.BlockSpec((RB, 128), lambda i: (i, 0)),
        ],
        out_shape=[jax.ShapeDtypeStruct((PACK, 128), jnp.float32)] * 2,
    )(aggp, h, dis, inv, bt, Wbd)


def _k4_body(a_ref, h2_ref, dis_ref, inv_ref, b2_ref, wc_ref, bc_ref,
             out_ref, emb_ref):
    c2 = jnp.tanh(
        dis_ref[...] * (a_ref[0] + a_ref[1])
        + h2_ref[...] * inv_ref[...]
        + b2_ref[...]
    )
    out_ref[...] = (
        jnp.dot(c2, wc_ref[...], preferred_element_type=jnp.float32) + bc_ref[...]
    )
    emb_ref[...] = jnp.concatenate([c2[:, 0:EMB], c2[:, H:H + EMB]], axis=1)


def _tc_final(aggp, h2, dis, inv, b2t, Wcbd, bct):
    return pl.pallas_call(
        _k4_body,
        grid=(GRID,),
        in_specs=[
            pl.BlockSpec((NC, RB, 128), lambda i: (0, i, 0)),
            pl.BlockSpec((RB, 128), lambda i: (i, 0)),
            pl.BlockSpec((RB, 128), lambda i: (i, 0)),
            pl.BlockSpec((RB, 128), lambda i: (i, 0)),
            pl.BlockSpec((1, 128), lambda i: (0, 0)),
            pl.BlockSpec((128, 2 * NCLS), lambda i: (0, 0)),
            pl.BlockSpec((1, 2 * NCLS), lambda i: (0, 0)),
        ],
        out_specs=[
            pl.BlockSpec((RB, 2 * NCLS), lambda i: (i, 0)),
            pl.BlockSpec((RB, 2 * EMB), lambda i: (i, 0)),
        ],
        out_shape=[
            jax.ShapeDtypeStruct((PACK, 2 * NCLS), jnp.float32),
            jax.ShapeDtypeStruct((PACK, 2 * EMB), jnp.float32),
        ],
    )(aggp, h2, dis, inv, b2t, Wcbd, bct)


def _blockdiag(W):
    k, m = W.shape
    z = jnp.zeros((k, m), jnp.float32)
    return jnp.concatenate(
        [jnp.concatenate([W, z], axis=1), jnp.concatenate([z, W], axis=1)], axis=0
    )


def kernel(x, edge_index, W0, b0, W1, b1, W2, b2, Wc, bc):
    ei = edge_index.astype(jnp.int32)
    pad = FBLK * BLK - E
    junk = JUNK + jnp.arange(pad, dtype=jnp.int32) % (NACC - N)
    srcpad = jnp.arange(pad, dtype=jnp.int32) * 79 % N
    src = jnp.concatenate([ei[0], srcpad]).reshape(FBLK, BLK)
    dst = jnp.concatenate([ei[1], junk]).reshape(FBLK, BLK)

    xp = jnp.concatenate(
        [x, jnp.zeros((NACC - N, D_IN), jnp.float32)]
    ).reshape(PACK, 2 * D_IN)

    degp = _sc_hist(dst).reshape(NC, PACK, 128)

    h0, hs0, dis, inv = _tc_prep(xp, _blockdiag(W0), degp)

    a0 = _sc_agg(hs0.reshape(NACC, H), src, dst).reshape(NC, PACK, 128)
    h1, hs1 = _tc_mid(False, a0, h0, dis, inv,
                      jnp.tile(b0, 2).reshape(1, 128), _blockdiag(W1))

    a1 = _sc_agg(hs1.reshape(NACC, H), src, dst).reshape(NC, PACK, 128)
    W2p = jnp.concatenate([W2, jnp.zeros((H, H - EMB), jnp.float32)], axis=1)
    h2, hs2 = _tc_mid(True, a1, h1, dis, inv,
                      jnp.tile(b1, 2).reshape(1, 128), _blockdiag(W2p))

    a2 = _sc_agg(hs2.reshape(NACC, H), src, dst).reshape(NC, PACK, 128)
    b2p = jnp.concatenate([b2, jnp.zeros((H - EMB,), jnp.float32)])
    Wcp = jnp.concatenate([Wc, jnp.zeros((H - EMB, NCLS), jnp.float32)], axis=0)
    out_pk, emb_pk = _tc_final(
        a2, h2, dis, inv,
        jnp.tile(b2p, 2).reshape(1, 128), _blockdiag(Wcp),
        jnp.tile(bc, 2).reshape(1, 2 * NCLS),
    )

    out = out_pk.reshape(NACC, NCLS)[:N]
    emb = emb_pk.reshape(NACC, EMB)[:N]
    return (out, emb)


# speedup vs baseline: 1.5007x; 1.0014x over previous
import functools

import jax
import jax.numpy as jnp
from jax import lax
from jax.experimental import pallas as pl
from jax.experimental.pallas import tpu as pltpu
from jax.experimental.pallas import tpu_sc as plsc

N = 10000
E = 320000
D_IN = 128
H = 64
EMB = 2
NCLS = 4

NC = 2
NS = 16
NW = NC * NS
LANES = 16
BLK = 128
BPW = 80
NBLK = NW * BPW
EPAD = NBLK * BLK
FBLK = NBLK + 104
NACC = 10240
PACK = NACC // 2
RPS = NACC // NS
JUNK = N

NBUF = 4
BPW0 = 80
BPW1 = 2 * BPW - BPW0
MAXB = max(BPW0, BPW1)

_mesh = plsc.VectorSubcoreMesh(core_axis_name="c", subcore_axis_name="s")


@functools.partial(
    pl.kernel,
    out_type=jax.ShapeDtypeStruct((NC, NACC, H), jnp.float32),
    mesh=_mesh,
    compiler_params=pltpu.CompilerParams(use_tc_tiling_on_sc=False),
    scratch_types=[
        pltpu.VMEM((MAXB, BLK), jnp.int32),
        pltpu.VMEM((MAXB, BLK), jnp.int32),
        [pltpu.VMEM((BLK, H), jnp.float32) for _ in range(NBUF)],
        pltpu.VMEM_SHARED((NACC, H), jnp.float32),
        [pltpu.SemaphoreType.DMA for _ in range(NBUF)],
        [pltpu.SemaphoreType.DMA for _ in range(NBUF)],
    ],
)
def _sc_agg(hs_hbm, src_hbm, dst_hbm, out_hbm, sidx, didx, rows, acc, gsem, ssem):
    c = lax.axis_index("c")
    s = lax.axis_index("s")
    start = s * (2 * BPW) + c * BPW0
    nblk = jnp.where(c == 0, BPW0, BPW1)

    def g_start(b, j):
        pltpu.async_copy(hs_hbm.at[sidx.at[b]], rows[j], gsem[j])

    def g_wait(j):
        pltpu.make_async_copy(hs_hbm.at[pl.ds(0, BLK)], rows[j], gsem[j]).wait()

    def s_start(b, j):
        pltpu.async_copy(rows[j], acc.at[didx.at[b]], ssem[j], add=True)

    def s_wait(j):
        pltpu.make_async_copy(rows[j], acc.at[pl.ds(0, BLK)], ssem[j]).wait()

    @pl.loop(0, BLK)
    def _(r):
        @pl.loop(0, H, step=LANES)
        def _(k):
            rows[0][r, pl.ds(k, LANES)] = jnp.zeros((LANES,), jnp.float32)

    @pl.loop(0, RPS // BLK)
    def _(j):
        pltpu.sync_copy(rows[0], acc.at[pl.ds(s * RPS + j * BLK, BLK)])

    pltpu.sync_copy(src_hbm.at[pl.ds(start, MAXB)], sidx)
    pltpu.sync_copy(dst_hbm.at[pl.ds(start, MAXB)], didx)
    plsc.subcore_barrier()

    for j in range(NBUF):
        g_start(j, j)

    @pl.loop(0, nblk - NBUF, step=NBUF)
    def _(b0):
        for j in range(NBUF):
            g_wait(j)
            s_start(b0 + j, j)
        for j in range(NBUF):
            s_wait(j)
            g_start(b0 + NBUF + j, j)

    for j in range(NBUF):
        g_wait(j)
        s_start(nblk - NBUF + j, j)
    for j in range(NBUF):
        s_wait(j)

    plsc.subcore_barrier()
    pltpu.sync_copy(
        acc.at[pl.ds(s * RPS, RPS)],
        out_hbm.at[c].at[pl.ds(s * RPS, RPS)],
    )


@functools.partial(
    pl.kernel,
    out_type=jax.ShapeDtypeStruct((NC, NACC, H), jnp.float32),
    mesh=_mesh,
    compiler_params=pltpu.CompilerParams(use_tc_tiling_on_sc=False),
    scratch_types=[
        pltpu.VMEM((BPW, BLK), jnp.int32),
        pltpu.VMEM((BLK, LANES), jnp.float32),
        pltpu.VMEM((RPS, LANES), jnp.float32),
        pltpu.VMEM((RPS, H), jnp.float32),
        pltpu.VMEM_SHARED((NACC, LANES), jnp.float32),
        pltpu.SemaphoreType.DMA,
    ],
)
def _sc_hist(dst_hbm, out_hbm, didx, ones, t16, t64, acc, hsem):
    c = lax.axis_index("c")
    s = lax.axis_index("s")
    wid = c * NS + s

    @pl.loop(0, BLK)
    def _(r):
        ones[r, pl.ds(0, LANES)] = jnp.zeros((LANES,), jnp.float32)

    @pl.loop(0, RPS // BLK)
    def _(j):
        pltpu.sync_copy(ones, acc.at[pl.ds(s * RPS + j * BLK, BLK)])

    @pl.loop(0, BLK)
    def _(r):
        ones[r, pl.ds(0, LANES)] = jnp.full((LANES,), 1.0, jnp.float32)

    pltpu.sync_copy(dst_hbm.at[pl.ds(wid * BPW, BPW)], didx)
    plsc.subcore_barrier()

    @pl.loop(0, BPW)
    def _(b):
        pltpu.async_copy(ones, acc.at[didx.at[b]], hsem, add=True)

    @pl.loop(0, BPW)
    def _(b):
        pltpu.make_async_copy(ones, acc.at[pl.ds(0, BLK)], hsem).wait()

    plsc.subcore_barrier()
    pltpu.sync_copy(acc.at[pl.ds(s * RPS, RPS)], t16)

    @pl.loop(0, RPS)
    def _(r):
        v = t16[r, pl.ds(0, LANES)]
        @pl.loop(0, H, step=LANES)
        def _(k):
            t64[r, pl.ds(k, LANES)] = v

    pltpu.sync_copy(t64, out_hbm.at[c].at[pl.ds(s * RPS, RPS)])


RB = PACK // 5
GRID = 5


def _k1_body(x_ref, w0_ref, dg_ref, h0_ref, hs0_ref, dis_ref, inv_ref):
    deg = dg_ref[0] + dg_ref[1] + 1.0
    dis = lax.rsqrt(deg)
    inv = 1.0 / deg
    h0 = jnp.dot(x_ref[...], w0_ref[...], preferred_element_type=jnp.float32)
    h0_ref[...] = h0
    hs0_ref[...] = h0 * dis
    dis_ref[...] = dis
    inv_ref[...] = inv


def _tc_prep(xp, W0bd, degp):
    return pl.pallas_call(
        _k1_body,
        grid=(GRID,),
        in_specs=[
            pl.BlockSpec((RB, 2 * D_IN), lambda i: (i, 0)),
            pl.BlockSpec((2 * D_IN, 128), lambda i: (0, 0)),
            pl.BlockSpec((NC, RB, 128), lambda i: (0, i, 0)),
        ],
        out_specs=[pl.BlockSpec((RB, 128), lambda i: (i, 0))] * 4,
        out_shape=[jax.ShapeDtypeStruct((PACK, 128), jnp.float32)] * 4,
    )(xp, W0bd, degp)


def _mid_body(act, a_ref, h_ref, dis_ref, inv_ref, b_ref, w_ref, hn_ref, hsn_ref):
    c = dis_ref[...] * (a_ref[0] + a_ref[1]) + h_ref[...] * inv_ref[...] + b_ref[...]
    if act:
        c = jnp.tanh(c)
    hn = jnp.dot(c, w_ref[...], preferred_element_type=jnp.float32)
    hn_ref[...] = hn
    hsn_ref[...] = hn * dis_ref[...]


def _tc_mid(act, aggp, h, dis, inv, bt, Wbd):
    return pl.pallas_call(
        functools.partial(_mid_body, act),
        grid=(GRID,),
        in_specs=[
            pl.BlockSpec((NC, RB, 128), lambda i: (0, i, 0)),
            pl.BlockSpec((RB, 128), lambda i: (i, 0)),
            pl.BlockSpec((RB, 128), lambda i: (i, 0)),
            pl.BlockSpec((RB, 128), lambda i: (i, 0)),
            pl.BlockSpec((1, 128), lambda i: (0, 0)),
            pl.BlockSpec((128, 128), lambda i: (0, 0)),
        ],
        out_specs=[
            pl.BlockSpec((RB, 128), lambda i: (i, 0)),
            pl.BlockSpec((RB, 128), lambda i: (i, 0)),
        ],
        out_shape=[jax.ShapeDtypeStruct((PACK, 128), jnp.float32)] * 2,
    )(aggp, h, dis, inv, bt, Wbd)


def _k4_body(a_ref, h2_ref, dis_ref, inv_ref, b2_ref, wc_ref, bc_ref,
             out_ref, emb_ref):
    c2 = jnp.tanh(
        dis_ref[...] * (a_ref[0] + a_ref[1])
        + h2_ref[...] * inv_ref[...]
        + b2_ref[...]
    )
    out_ref[...] = (
        jnp.dot(c2, wc_ref[...], preferred_element_type=jnp.float32) + bc_ref[...]
    )
    emb_ref[...] = jnp.concatenate([c2[:, 0:EMB], c2[:, H:H + EMB]], axis=1)


def _tc_final(aggp, h2, dis, inv, b2t, Wcbd, bct):
    return pl.pallas_call(
        _k4_body,
        grid=(GRID,),
        in_specs=[
            pl.BlockSpec((NC, RB, 128), lambda i: (0, i, 0)),
            pl.BlockSpec((RB, 128), lambda i: (i, 0)),
            pl.BlockSpec((RB, 128), lambda i: (i, 0)),
            pl.BlockSpec((RB, 128), lambda i: (i, 0)),
            pl.BlockSpec((1, 128), lambda i: (0, 0)),
            pl.BlockSpec((128, 2 * NCLS), lambda i: (0, 0)),
            pl.BlockSpec((1, 2 * NCLS), lambda i: (0, 0)),
        ],
        out_specs=[
            pl.BlockSpec((RB, 2 * NCLS), lambda i: (i, 0)),
            pl.BlockSpec((RB, 2 * EMB), lambda i: (i, 0)),
        ],
        out_shape=[
            jax.ShapeDtypeStruct((PACK, 2 * NCLS), jnp.float32),
            jax.ShapeDtypeStruct((PACK, 2 * EMB), jnp.float32),
        ],
    )(aggp, h2, dis, inv, b2t, Wcbd, bct)


def _blockdiag(W):
    k, m = W.shape
    z = jnp.zeros((k, m), jnp.float32)
    return jnp.concatenate(
        [jnp.concatenate([W, z], axis=1), jnp.concatenate([z, W], axis=1)], axis=0
    )


def kernel(x, edge_index, W0, b0, W1, b1, W2, b2, Wc, bc):
    ei = edge_index.astype(jnp.int32)
    pad = FBLK * BLK - E
    junk = JUNK + jnp.arange(pad, dtype=jnp.int32) % (NACC - N)
    srcpad = jnp.arange(pad, dtype=jnp.int32) * 79 % N
    src = jnp.concatenate([ei[0], srcpad]).reshape(FBLK, BLK)
    dst = jnp.concatenate([ei[1], junk]).reshape(FBLK, BLK)

    xp = jnp.concatenate(
        [x, jnp.zeros((NACC - N, D_IN), jnp.float32)]
    ).reshape(PACK, 2 * D_IN)

    degp = _sc_hist(dst).reshape(NC, PACK, 128)

    h0, hs0, dis, inv = _tc_prep(xp, _blockdiag(W0), degp)

    a0 = _sc_agg(hs0.reshape(NACC, H), src, dst).reshape(NC, PACK, 128)
    h1, hs1 = _tc_mid(False, a0, h0, dis, inv,
                      jnp.tile(b0, 2).reshape(1, 128), _blockdiag(W1))

    a1 = _sc_agg(hs1.reshape(NACC, H), src, dst).reshape(NC, PACK, 128)
    W2p = jnp.concatenate([W2, jnp.zeros((H, H - EMB), jnp.float32)], axis=1)
    h2, hs2 = _tc_mid(True, a1, h1, dis, inv,
                      jnp.tile(b1, 2).reshape(1, 128), _blockdiag(W2p))

    a2 = _sc_agg(hs2.reshape(NACC, H), src, dst).reshape(NC, PACK, 128)
    b2p = jnp.concatenate([b2, jnp.zeros((H - EMB,), jnp.float32)])
    Wcp = jnp.concatenate([Wc, jnp.zeros((H - EMB, NCLS), jnp.float32)], axis=0)
    out_pk, emb_pk = _tc_final(
        a2, h2, dis, inv,
        jnp.tile(b2p, 2).reshape(1, 128), _blockdiag(Wcp),
        jnp.tile(bc, 2).reshape(1, 2 * NCLS),
    )

    out = out_pk.reshape(NACC, NCLS)[:N]
    emb = emb_pk.reshape(NACC, EMB)[:N]
    return (out, emb)
